# baseline (speedup 1.0000x reference)
import functools

import jax
import jax.numpy as jnp
from jax import lax
from jax.experimental import pallas as pl
from jax.experimental.pallas import tpu as pltpu
from jax.experimental.pallas import tpu_sc as plsc

GRID = 16
M = GRID * GRID * GRID
N = 32768
NRAYS = 1024
NSAMP = 32
C = 40
ROWS = M * C
L_XYZ = 10
L_DIR = 4
NW = 32
CH = N // NW
MB = 32
EP_W = 64
ED_W = 32
Y_W = 16


def _encode_body(nlev, x_ref, o_ref):
    p = x_ref[...]
    s = jnp.sin(p)
    c = jnp.cos(p)
    o_ref[pl.ds(0, 3), :] = p[0:3, :]
    for i in range(nlev):
        o_ref[pl.ds(3 + 6 * i, 3), :] = s[0:3, :]
        o_ref[pl.ds(6 + 6 * i, 3), :] = c[0:3, :]
        if i + 1 < nlev:
            s2 = 2.0 * s * c
            c2 = 1.0 - 2.0 * s * s
            s, c = s2, c2
    zrow = jnp.zeros((o_ref.shape[0] - (3 + 6 * nlev), p.shape[1]), jnp.float32)
    o_ref[pl.ds(3 + 6 * nlev, o_ref.shape[0] - (3 + 6 * nlev)), :] = zrow


def _encode_pts(x8):
    blk = 2048
    return pl.pallas_call(
        functools.partial(_encode_body, L_XYZ),
        grid=(N // blk,),
        in_specs=[pl.BlockSpec((8, blk), lambda i: (0, i))],
        out_specs=pl.BlockSpec((EP_W, blk), lambda i: (0, i)),
        out_shape=jax.ShapeDtypeStruct((EP_W, N), jnp.float32),
    )(x8)


def _encode_dirs(d8):
    return pl.pallas_call(
        functools.partial(_encode_body, L_DIR),
        grid=(1,),
        in_specs=[pl.BlockSpec((8, NRAYS), lambda i: (0, 0))],
        out_specs=pl.BlockSpec((ED_W, NRAYS), lambda i: (0, 0)),
        out_shape=jax.ShapeDtypeStruct((ED_W, NRAYS), jnp.float32),
    )(d8)


@functools.cache
def _mesh():
    return plsc.VectorSubcoreMesh(core_axis_name="c", subcore_axis_name="s")


_SC_PARAMS = pltpu.CompilerParams(use_tc_tiling_on_sc=False,
                                  needs_layout_passes=False)


def _wid():
    return lax.axis_index("c") * 16 + lax.axis_index("s")


IOTA16 = lambda: lax.iota(jnp.int32, 16)


def _rank_last(m16, scr16):
    iota = IOTA16()
    sk, sl = plsc.sort_key_val(m16, iota)
    scr16[...] = sk
    prevk = plsc.load_gather(scr16, [jnp.maximum(iota - 1, 0)])
    nextk = plsc.load_gather(scr16, [jnp.minimum(iota + 1, 15)])
    isb = (iota == 0) | (prevk != sk)
    start = plsc.cummax(jnp.where(isb, iota, 0))
    rank_sorted = iota - start
    last_sorted = (iota == 15) | (nextk != sk)
    plsc.store_scatter(scr16, [sl], rank_sorted)
    rank = scr16[...]
    plsc.store_scatter(scr16, [sl], last_sorted.astype(jnp.int32))
    last = scr16[...] != 0
    return rank, last


def _b1_body(xs_h, ys_h, zs_h, m_h, s1_h, tot_h, xv, yv, zv, mv, sv, hist,
             scr16):
    w = _wid()
    base = w * CH
    pltpu.sync_copy(xs_h.at[pl.ds(base, CH)], xv)
    pltpu.sync_copy(ys_h.at[pl.ds(base, CH)], yv)
    pltpu.sync_copy(zs_h.at[pl.ds(base, CH)], zv)

    def zero_body(t, _):
        hist[pl.ds(t * 16, 16)] = jnp.zeros((16,), jnp.int32)
        return 0
    lax.fori_loop(0, M // 16, zero_body, 0)

    def body(j, _):
        off = j * 16
        x16 = xv[pl.ds(off, 16)]
        y16 = yv[pl.ds(off, 16)]
        z16 = zv[pl.ds(off, 16)]
        def vox(v):
            return jnp.clip(v * float(GRID), 0.0, float(GRID - 1)).astype(jnp.int32)
        m16 = vox(x16) * (GRID * GRID) + vox(y16) * GRID + vox(z16)
        mv[pl.ds(off, 16)] = m16
        rank, lastm = _rank_last(m16, scr16)
        old = plsc.load_gather(hist, [m16])
        sv[pl.ds(off, 16)] = old + rank
        plsc.store_scatter(hist, [m16], old + rank + 1, mask=lastm)
        return 0
    lax.fori_loop(0, CH // 16, body, 0)

    pltpu.sync_copy(mv, m_h.at[w])
    pltpu.sync_copy(sv, s1_h.at[w])
    pltpu.sync_copy(hist, tot_h.at[w])


def _b1(xs, ys, zs):
    return pl.kernel(
        _b1_body,
        out_type=[
            jax.ShapeDtypeStruct((NW, CH), jnp.int32),
            jax.ShapeDtypeStruct((NW, CH), jnp.int32),
            jax.ShapeDtypeStruct((NW, M), jnp.int32),
        ],
        mesh=_mesh(),
        compiler_params=_SC_PARAMS,
        scratch_types=[
            pltpu.VMEM((CH,), jnp.float32),
            pltpu.VMEM((CH,), jnp.float32),
            pltpu.VMEM((CH,), jnp.float32),
            pltpu.VMEM((CH,), jnp.int32),
            pltpu.VMEM((CH,), jnp.int32),
            pltpu.VMEM((M,), jnp.int32),
            pltpu.VMEM((16,), jnp.int32),
        ],
    )(xs, ys, zs)


def _b15_body(tot_h, pre_h, buf, prebuf, sem):
    u = _wid()
    mpw = M // NW
    copies = [pltpu.make_async_copy(
        tot_h.at[w2, pl.ds(u * mpw, mpw)], buf.at[w2], sem)
        for w2 in range(NW)]
    for cp in copies:
        cp.start()
    for cp in copies:
        cp.wait()

    def g_body(g, _):
        def w_body(w2, run):
            prebuf[w2, pl.ds(g * 16, 16)] = run
            return run + plsc.load_gather(
                buf, [jnp.full((16,), w2, jnp.int32), IOTA16() + g * 16])
        lax.fori_loop(0, NW, w_body, jnp.zeros((16,), jnp.int32))
        return 0
    lax.fori_loop(0, mpw // 16, g_body, 0)

    pltpu.sync_copy(prebuf, pre_h.at[:, pl.ds(u * mpw, mpw)])


def _b15(tot):
    return pl.kernel(
        _b15_body,
        out_type=jax.ShapeDtypeStruct((NW, M), jnp.int32),
        mesh=_mesh(),
        compiler_params=_SC_PARAMS,
        scratch_types=[
            pltpu.VMEM((NW, M // NW), jnp.int32),
            pltpu.VMEM((NW, M // NW), jnp.int32),
            pltpu.SemaphoreType.DMA,
        ],
    )(tot)


def _b2_body(m_h, s1_h, pre_h, ep_h, ed_h, xp_h, xd_h, dest_h,
             acc, mv, sv, dv, epb, edb, sem):
    w = _wid()
    pltpu.sync_copy(pre_h.at[w], acc)
    pltpu.sync_copy(m_h.at[w], mv)
    pltpu.sync_copy(s1_h.at[w], sv)

    def body(j, _):
        off = j * 16
        m16 = mv[pl.ds(off, 16)]
        slot = sv[pl.ds(off, 16)] + plsc.load_gather(acc, [m16])
        slot = jnp.minimum(slot, C - 1)
        dest16 = m16 * C + slot
        dv[j // 8, pl.ds((j % 8) * 16, 16)] = dest16
        return 0
    lax.fori_loop(0, CH // 16, body, 0)

    pltpu.sync_copy(dv, dest_h.at[w])
    base = w * CH
    pltpu.sync_copy(ep_h.at[pl.ds(base, CH), :], epb)
    pltpu.sync_copy(ed_h.at[pl.ds(base, CH), :], edb)
    copies = []
    for t in range(CH // 128):
        copies.append(pltpu.make_async_copy(
            epb.at[pl.ds(t * 128, 128), :], xp_h.at[dv.at[t]], sem))
        copies.append(pltpu.make_async_copy(
            edb.at[pl.ds(t * 128, 128), :], xd_h.at[dv.at[t]], sem))
    for cp in copies:
        cp.start()
    for cp in copies:
        cp.wait()


def _b2(m_all, s1, pre, ep, ed):
    return pl.kernel(
        _b2_body,
        out_type=[
            jax.ShapeDtypeStruct((ROWS, EP_W), jnp.bfloat16),
            jax.ShapeDtypeStruct((ROWS, ED_W), jnp.bfloat16),
            jax.ShapeDtypeStruct((NW, CH // 128, 128), jnp.int32),
        ],
        mesh=_mesh(),
        compiler_params=_SC_PARAMS,
        scratch_types=[
            pltpu.VMEM((M,), jnp.int32),
            pltpu.VMEM((CH,), jnp.int32),
            pltpu.VMEM((CH,), jnp.int32),
            pltpu.VMEM((CH // 128, 128), jnp.int32),
            pltpu.VMEM((CH, EP_W), jnp.bfloat16),
            pltpu.VMEM((CH, ED_W), jnp.bfloat16),
            pltpu.SemaphoreType.DMA,
        ],
    )(m_all, s1, pre, ep, ed)


def _c_body(xp_ref, xd_ref, w0_ref, b0_ref, w1_ref, b1_ref, fw_ref, fb_ref,
            vw_ref, vb_ref, yw_ref, yb_ref, y_ref):
    f32 = jnp.float32
    bf16 = jnp.bfloat16
    dims_t = (((1,), (1,)), ((), ()))
    dims_n = (((1,), (0,)), ((), ()))

    X = xp_ref[...]
    X63 = X[:, 0:63]
    XD = xd_ref[...]

    def rep(b_ref, width):
        b = b_ref[...]
        return jnp.broadcast_to(b[:, None, :], (MB, C, width)).reshape(MB * C, width)

    def layer_v(hval, w_ref2, dims):
        wb = w_ref2[...].astype(bf16)
        outs = [lax.dot_general(hval[i * C:(i + 1) * C, :], wb[i], dims,
                                preferred_element_type=f32)
                for i in range(MB)]
        return jnp.concatenate(outs, axis=0)

    H1 = jax.nn.relu(layer_v(X63, w0_ref, dims_t) + rep(b0_ref, 32)).astype(bf16)
    H2 = jax.nn.relu(layer_v(H1, w1_ref, dims_t) + rep(b1_ref, 32)).astype(bf16)
    FT = (layer_v(H2, fw_ref, dims_t) + rep(fb_ref, 32)).astype(bf16)
    HVin = jnp.concatenate([FT, XD[:, 0:27]], axis=1)
    HV = jax.nn.relu(layer_v(HVin, vw_ref, dims_t) + rep(vb_ref, 32)).astype(bf16)
    YC = jnp.concatenate([HV, H2], axis=1)
    Y = layer_v(YC, yw_ref, dims_n) + rep(yb_ref, Y_W)
    y_ref[...] = Y


def _c(xp, xd, w0, b0, w1, b1, fw, fb, vw, vb, yw, yb):
    nsteps = M // MB
    mspec = lambda shp: pl.BlockSpec((MB,) + shp, lambda i: (i,) + (0,) * len(shp))
    return pl.pallas_call(
        _c_body,
        grid=(nsteps,),
        in_specs=[
            pl.BlockSpec((MB * C, EP_W), lambda i: (i, 0)),
            pl.BlockSpec((MB * C, ED_W), lambda i: (i, 0)),
            mspec((32, 63)), mspec((32,)),
            mspec((32, 32)), mspec((32,)),
            mspec((32, 32)), mspec((32,)),
            mspec((32, 59)), mspec((32,)),
            mspec((64, Y_W)), mspec((Y_W,)),
        ],
        out_specs=pl.BlockSpec((MB * C, Y_W), lambda i: (i, 0)),
        out_shape=jax.ShapeDtypeStruct((ROWS, Y_W), jnp.float32),
    )(xp, xd, w0, b0, w1, b1, fw, fb, vw, vb, yw, yb)


def _d_body(y_h, dest_h, out_h, dv, yb, sem):
    w = _wid()
    pltpu.sync_copy(dest_h.at[w], dv)
    for t in range(CH // 128):
        cp = pltpu.make_async_copy(y_h.at[dv.at[t]], yb, sem)
        cp.start()
        cp.wait()
        pltpu.sync_copy(yb, out_h.at[pl.ds(w * CH + t * 128, 128), :])


def _d(y, dest):
    return pl.kernel(
        _d_body,
        out_type=jax.ShapeDtypeStruct((N, Y_W), jnp.float32),
        mesh=_mesh(),
        compiler_params=_SC_PARAMS,
        scratch_types=[
            pltpu.VMEM((CH // 128, 128), jnp.int32),
            pltpu.VMEM((128, Y_W), jnp.float32),
            pltpu.SemaphoreType.DMA,
        ],
    )(y, dest)


def kernel(pts, viewdirs, pts_w0, pts_b0, pts_w1, pts_b1, feat_w, feat_b,
           sigma_w, sigma_b, view_w, view_b, rgb_w, rgb_b):
    pts_flat = pts.reshape(N, 3)
    x8 = jnp.pad(pts_flat.T, ((0, 5), (0, 0)))
    d8 = jnp.pad(viewdirs.T, ((0, 5), (0, 0)))

    eT = _encode_pts(x8)
    edT = _encode_dirs(d8)
    ep = eT.T.astype(jnp.bfloat16)
    ed = jnp.repeat(edT.T.astype(jnp.bfloat16), NSAMP, axis=0)

    m_all, s1, tot = _b1(x8[0], x8[1], x8[2])
    pre = _b15(tot)
    xp, xd, dest = _b2(m_all, s1, pre, ep, ed)

    rgb_wT = jnp.transpose(rgb_w, (0, 2, 1))
    top = jnp.pad(rgb_wT, ((0, 0), (0, 0), (0, Y_W - 3)))
    bot = jnp.pad(sigma_w.reshape(M, 32, 1), ((0, 0), (0, 0), (3, Y_W - 4)))
    yw = jnp.concatenate([top, bot], axis=1)
    yb = (jnp.pad(rgb_b, ((0, 0), (0, Y_W - 3)))
          + jnp.pad(sigma_b, ((0, 0), (3, Y_W - 4))))

    y = _c(xp, xd, pts_w0, pts_b0, pts_w1, pts_b1, feat_w, feat_b,
           view_w, view_b, yw, yb)
    res = _d(y, dest)

    rgb = res[:, 0:3].reshape(NRAYS, NSAMP, 3)
    sigma = res[:, 3:4].reshape(NRAYS, NSAMP, 1)
    return (rgb, sigma)


# speedup vs baseline: 1.0652x; 1.0652x over previous
import functools

import jax
import jax.numpy as jnp
from jax import lax
from jax.experimental import pallas as pl
from jax.experimental.pallas import tpu as pltpu
from jax.experimental.pallas import tpu_sc as plsc

GRID = 16
M = GRID * GRID * GRID
N = 32768
NRAYS = 1024
NSAMP = 32
C = 40
ROWS = M * C
L_XYZ = 10
L_DIR = 4
NW = 32
CH = N // NW
MB = 32
EP_W = 64
ED_W = 32
Y_W = 16


def _encode_body(nlev, x_ref, o_ref):
    p = x_ref[...]
    s = jnp.sin(p)
    c = jnp.cos(p)
    o_ref[pl.ds(0, 3), :] = p[0:3, :]
    for i in range(nlev):
        o_ref[pl.ds(3 + 6 * i, 3), :] = s[0:3, :]
        o_ref[pl.ds(6 + 6 * i, 3), :] = c[0:3, :]
        if i + 1 < nlev:
            s2 = 2.0 * s * c
            c2 = 1.0 - 2.0 * s * s
            s, c = s2, c2
    zrow = jnp.zeros((o_ref.shape[0] - (3 + 6 * nlev), p.shape[1]), jnp.float32)
    o_ref[pl.ds(3 + 6 * nlev, o_ref.shape[0] - (3 + 6 * nlev)), :] = zrow


def _encode_pts(x8):
    blk = 2048
    return pl.pallas_call(
        functools.partial(_encode_body, L_XYZ),
        grid=(N // blk,),
        in_specs=[pl.BlockSpec((8, blk), lambda i: (0, i))],
        out_specs=pl.BlockSpec((EP_W, blk), lambda i: (0, i)),
        out_shape=jax.ShapeDtypeStruct((EP_W, N), jnp.float32),
    )(x8)


def _encode_dirs(d8):
    return pl.pallas_call(
        functools.partial(_encode_body, L_DIR),
        grid=(1,),
        in_specs=[pl.BlockSpec((8, NRAYS), lambda i: (0, 0))],
        out_specs=pl.BlockSpec((ED_W, NRAYS), lambda i: (0, 0)),
        out_shape=jax.ShapeDtypeStruct((ED_W, NRAYS), jnp.float32),
    )(d8)


@functools.cache
def _mesh():
    return plsc.VectorSubcoreMesh(core_axis_name="c", subcore_axis_name="s")


_SC_PARAMS = pltpu.CompilerParams(use_tc_tiling_on_sc=False,
                                  needs_layout_passes=False)


def _wid():
    return lax.axis_index("c") * 16 + lax.axis_index("s")


IOTA16 = lambda: lax.iota(jnp.int32, 16)


def _rank_last(m16, scr16):
    iota = IOTA16()
    sk, sl = plsc.sort_key_val(m16, iota)
    scr16[...] = sk
    prevk = plsc.load_gather(scr16, [jnp.maximum(iota - 1, 0)])
    nextk = plsc.load_gather(scr16, [jnp.minimum(iota + 1, 15)])
    isb = (iota == 0) | (prevk != sk)
    start = plsc.cummax(jnp.where(isb, iota, 0))
    rank_sorted = iota - start
    last_sorted = (iota == 15) | (nextk != sk)
    plsc.store_scatter(scr16, [sl], rank_sorted)
    rank = scr16[...]
    plsc.store_scatter(scr16, [sl], last_sorted.astype(jnp.int32))
    last = scr16[...] != 0
    return rank, last


def _b1_body(xs_h, ys_h, zs_h, m_h, s1_h, tot_h, xv, yv, zv, mv, sv, hist,
             scr16):
    w = _wid()
    base = w * CH
    pltpu.sync_copy(xs_h.at[pl.ds(base, CH)], xv)
    pltpu.sync_copy(ys_h.at[pl.ds(base, CH)], yv)
    pltpu.sync_copy(zs_h.at[pl.ds(base, CH)], zv)

    def zero_body(t, _):
        hist[pl.ds(t * 16, 16)] = jnp.zeros((16,), jnp.int32)
        return 0
    lax.fori_loop(0, M // 16, zero_body, 0)

    def body(j, _):
        off = j * 16
        x16 = xv[pl.ds(off, 16)]
        y16 = yv[pl.ds(off, 16)]
        z16 = zv[pl.ds(off, 16)]
        def vox(v):
            return jnp.clip(v * float(GRID), 0.0, float(GRID - 1)).astype(jnp.int32)
        m16 = vox(x16) * (GRID * GRID) + vox(y16) * GRID + vox(z16)
        mv[pl.ds(off, 16)] = m16
        rank, lastm = _rank_last(m16, scr16)
        old = plsc.load_gather(hist, [m16])
        sv[pl.ds(off, 16)] = old + rank
        plsc.store_scatter(hist, [m16], old + rank + 1, mask=lastm)
        return 0
    lax.fori_loop(0, CH // 16, body, 0)

    pltpu.sync_copy(mv, m_h.at[w])
    pltpu.sync_copy(sv, s1_h.at[w])
    pltpu.sync_copy(hist, tot_h.at[w])


def _b1(xs, ys, zs):
    return pl.kernel(
        _b1_body,
        out_type=[
            jax.ShapeDtypeStruct((NW, CH), jnp.int32),
            jax.ShapeDtypeStruct((NW, CH), jnp.int32),
            jax.ShapeDtypeStruct((NW, M), jnp.int32),
        ],
        mesh=_mesh(),
        compiler_params=_SC_PARAMS,
        scratch_types=[
            pltpu.VMEM((CH,), jnp.float32),
            pltpu.VMEM((CH,), jnp.float32),
            pltpu.VMEM((CH,), jnp.float32),
            pltpu.VMEM((CH,), jnp.int32),
            pltpu.VMEM((CH,), jnp.int32),
            pltpu.VMEM((M,), jnp.int32),
            pltpu.VMEM((16,), jnp.int32),
        ],
    )(xs, ys, zs)


def _b15_body(tot_h, pre_h, buf, prebuf, sem):
    u = _wid()
    mpw = M // NW
    copies = [pltpu.make_async_copy(
        tot_h.at[w2, pl.ds(u * mpw, mpw)], buf.at[w2], sem)
        for w2 in range(NW)]
    for cp in copies:
        cp.start()
    for cp in copies:
        cp.wait()

    def g_body(g, _):
        def w_body(w2, run):
            prebuf[w2, pl.ds(g * 16, 16)] = run
            return run + plsc.load_gather(
                buf, [jnp.full((16,), w2, jnp.int32), IOTA16() + g * 16])
        lax.fori_loop(0, NW, w_body, jnp.zeros((16,), jnp.int32))
        return 0
    lax.fori_loop(0, mpw // 16, g_body, 0)

    pltpu.sync_copy(prebuf, pre_h.at[:, pl.ds(u * mpw, mpw)])


def _b15(tot):
    return pl.kernel(
        _b15_body,
        out_type=jax.ShapeDtypeStruct((NW, M), jnp.int32),
        mesh=_mesh(),
        compiler_params=_SC_PARAMS,
        scratch_types=[
            pltpu.VMEM((NW, M // NW), jnp.int32),
            pltpu.VMEM((NW, M // NW), jnp.int32),
            pltpu.SemaphoreType.DMA,
        ],
    )(tot)


def _b2_body(m_h, s1_h, pre_h, ep_h, ed_h, xp_h, xd_h, dest_h,
             acc, mv, sv, dv, epb, edb, sem):
    w = _wid()
    pltpu.sync_copy(pre_h.at[w], acc)
    pltpu.sync_copy(m_h.at[w], mv)
    pltpu.sync_copy(s1_h.at[w], sv)

    def body(j, _):
        off = j * 16
        m16 = mv[pl.ds(off, 16)]
        slot = sv[pl.ds(off, 16)] + plsc.load_gather(acc, [m16])
        slot = jnp.minimum(slot, C - 1)
        dest16 = m16 * C + slot
        dv[j // 8, pl.ds((j % 8) * 16, 16)] = dest16
        return 0
    lax.fori_loop(0, CH // 16, body, 0)

    pltpu.sync_copy(dv, dest_h.at[w])
    base = w * CH
    pltpu.sync_copy(ep_h.at[pl.ds(base, CH), :], epb)
    pltpu.sync_copy(ed_h.at[pl.ds(base, CH), :], edb)
    copies = []
    for t in range(CH // 128):
        copies.append(pltpu.make_async_copy(
            epb.at[pl.ds(t * 128, 128), :], xp_h.at[dv.at[t]], sem))
        copies.append(pltpu.make_async_copy(
            edb.at[pl.ds(t * 128, 128), :], xd_h.at[dv.at[t]], sem))
    for cp in copies:
        cp.start()
    for cp in copies:
        cp.wait()


def _b2(m_all, s1, pre, ep, ed):
    return pl.kernel(
        _b2_body,
        out_type=[
            jax.ShapeDtypeStruct((ROWS, EP_W), jnp.bfloat16),
            jax.ShapeDtypeStruct((ROWS, ED_W), jnp.bfloat16),
            jax.ShapeDtypeStruct((NW, CH // 128, 128), jnp.int32),
        ],
        mesh=_mesh(),
        compiler_params=_SC_PARAMS,
        scratch_types=[
            pltpu.VMEM((M,), jnp.int32),
            pltpu.VMEM((CH,), jnp.int32),
            pltpu.VMEM((CH,), jnp.int32),
            pltpu.VMEM((CH // 128, 128), jnp.int32),
            pltpu.VMEM((CH, EP_W), jnp.bfloat16),
            pltpu.VMEM((CH, ED_W), jnp.bfloat16),
            pltpu.SemaphoreType.DMA,
        ],
    )(m_all, s1, pre, ep, ed)


def _c_body(xp_ref, xd_ref, w0_ref, b0_ref, w1_ref, b1_ref, fw_ref, fb_ref,
            vw_ref, vb_ref, yw_ref, yb_ref, y_ref):
    f32 = jnp.float32
    bf16 = jnp.bfloat16
    dims_t = (((1,), (1,)), ((), ()))
    dims_n = (((1,), (0,)), ((), ()))

    X = xp_ref[...]
    X63 = X[:, 0:63]
    XD = xd_ref[...]

    def rep(b_ref, width):
        b = b_ref[...]
        return jnp.broadcast_to(b[:, None, :], (MB, C, width)).reshape(MB * C, width)

    def layer_v(hval, w_ref2, dims):
        wb = w_ref2[...]
        outs = [lax.dot_general(hval[i * C:(i + 1) * C, :], wb[i], dims,
                                preferred_element_type=f32)
                for i in range(MB)]
        return jnp.concatenate(outs, axis=0)

    H1 = jax.nn.relu(layer_v(X63, w0_ref, dims_t) + rep(b0_ref, 32)).astype(bf16)
    H2 = jax.nn.relu(layer_v(H1, w1_ref, dims_t) + rep(b1_ref, 32)).astype(bf16)
    FT = (layer_v(H2, fw_ref, dims_t) + rep(fb_ref, 32)).astype(bf16)
    HVin = jnp.concatenate([FT, XD[:, 0:27]], axis=1)
    HV = jax.nn.relu(layer_v(HVin, vw_ref, dims_t) + rep(vb_ref, 32)).astype(bf16)
    YC = jnp.concatenate([HV, H2], axis=1)
    Y = layer_v(YC, yw_ref, dims_n) + rep(yb_ref, Y_W)
    y_ref[...] = Y


def _c(xp, xd, w0, b0, w1, b1, fw, fb, vw, vb, yw, yb):
    nsteps = M // MB
    mspec = lambda shp: pl.BlockSpec((MB,) + shp, lambda i: (i,) + (0,) * len(shp))
    return pl.pallas_call(
        _c_body,
        grid=(nsteps,),
        in_specs=[
            pl.BlockSpec((MB * C, EP_W), lambda i: (i, 0)),
            pl.BlockSpec((MB * C, ED_W), lambda i: (i, 0)),
            mspec((32, 63)), mspec((32,)),
            mspec((32, 32)), mspec((32,)),
            mspec((32, 32)), mspec((32,)),
            mspec((32, 59)), mspec((32,)),
            mspec((64, Y_W)), mspec((Y_W,)),
        ],
        out_specs=pl.BlockSpec((MB * C, Y_W), lambda i: (i, 0)),
        out_shape=jax.ShapeDtypeStruct((ROWS, Y_W), jnp.float32),
    )(xp, xd, w0, b0, w1, b1, fw, fb, vw, vb, yw, yb)


def _d_body(y_h, dest_h, out_h, dv, yb, sem):
    w = _wid()
    pltpu.sync_copy(dest_h.at[w], dv)
    for t in range(CH // 128):
        cp = pltpu.make_async_copy(y_h.at[dv.at[t]], yb, sem)
        cp.start()
        cp.wait()
        pltpu.sync_copy(yb, out_h.at[pl.ds(w * CH + t * 128, 128), :])


def _d(y, dest):
    return pl.kernel(
        _d_body,
        out_type=jax.ShapeDtypeStruct((N, Y_W), jnp.float32),
        mesh=_mesh(),
        compiler_params=_SC_PARAMS,
        scratch_types=[
            pltpu.VMEM((CH // 128, 128), jnp.int32),
            pltpu.VMEM((128, Y_W), jnp.float32),
            pltpu.SemaphoreType.DMA,
        ],
    )(y, dest)


def kernel(pts, viewdirs, pts_w0, pts_b0, pts_w1, pts_b1, feat_w, feat_b,
           sigma_w, sigma_b, view_w, view_b, rgb_w, rgb_b):
    pts_flat = pts.reshape(N, 3)
    x8 = jnp.pad(pts_flat.T, ((0, 5), (0, 0)))
    d8 = jnp.pad(viewdirs.T, ((0, 5), (0, 0)))

    eT = _encode_pts(x8)
    edT = _encode_dirs(d8)
    ep = eT.T.astype(jnp.bfloat16)
    ed = jnp.repeat(edT.T.astype(jnp.bfloat16), NSAMP, axis=0)

    m_all, s1, tot = _b1(x8[0], x8[1], x8[2])
    pre = _b15(tot)
    xp, xd, dest = _b2(m_all, s1, pre, ep, ed)

    bf16 = jnp.bfloat16
    rgb_wT = jnp.transpose(rgb_w.astype(bf16), (0, 2, 1))
    top = jnp.pad(rgb_wT, ((0, 0), (0, 0), (0, Y_W - 3)))
    bot = jnp.pad(sigma_w.astype(bf16).reshape(M, 32, 1),
                  ((0, 0), (0, 0), (3, Y_W - 4)))
    yw = jnp.concatenate([top, bot], axis=1)
    yb = (jnp.pad(rgb_b, ((0, 0), (0, Y_W - 3)))
          + jnp.pad(sigma_b, ((0, 0), (3, Y_W - 4))))

    y = _c(xp, xd, pts_w0.astype(bf16), pts_b0, pts_w1.astype(bf16), pts_b1,
           feat_w.astype(bf16), feat_b, view_w.astype(bf16), view_b, yw, yb)
    res = _d(y, dest)

    rgb = res[:, 0:3].reshape(NRAYS, NSAMP, 3)
    sigma = res[:, 3:4].reshape(NRAYS, NSAMP, 1)
    return (rgb, sigma)


# speedup vs baseline: 1.2359x; 1.1602x over previous
import functools

import jax
import jax.numpy as jnp
from jax import lax
from jax.experimental import pallas as pl
from jax.experimental.pallas import tpu as pltpu
from jax.experimental.pallas import tpu_sc as plsc

GRID = 16
M = GRID * GRID * GRID
N = 32768
NRAYS = 1024
NSAMP = 32
C = 40
ROWS = M * C
L_XYZ = 10
L_DIR = 4
NW = 32
CH = N // NW
MB = 32
EP_W = 64
ED_W = 32
XB_W = EP_W + ED_W
Y_W = 16


def _sincos_rows(p, nlev):
    s = jnp.sin(p)
    c = jnp.cos(p)
    pieces = [p[0:3, :]]
    for i in range(nlev):
        pieces.append(s[0:3, :])
        pieces.append(c[0:3, :])
        if i + 1 < nlev:
            s, c = 2.0 * s * c, 1.0 - 2.0 * s * s
    return pieces


def _encode_body(x_ref, d_ref, o_ref):
    blk = x_ref.shape[1]
    rblk = d_ref.shape[1]
    pieces = _sincos_rows(x_ref[...], L_XYZ)
    off = 0
    for pc in pieces:
        o_ref[pl.ds(off, 3), :] = pc
        off += 3
    o_ref[pl.ds(off, EP_W - off), :] = jnp.zeros((EP_W - off, blk), jnp.float32)
    dpieces = _sincos_rows(d_ref[...], L_DIR)
    dpieces.append(jnp.zeros((ED_W - 3 * (1 + 2 * L_DIR), rblk), jnp.float32))
    dire = jnp.concatenate(dpieces, axis=0)
    diree = jnp.broadcast_to(dire[:, :, None], (ED_W, rblk, NSAMP))
    o_ref[pl.ds(EP_W, ED_W), :] = diree.reshape(ED_W, blk)


def _encode(x8, d8):
    blk = 4096
    rblk = blk // NSAMP
    return pl.pallas_call(
        _encode_body,
        grid=(N // blk,),
        in_specs=[pl.BlockSpec((8, blk), lambda i: (0, i)),
                  pl.BlockSpec((8, rblk), lambda i: (0, i))],
        out_specs=pl.BlockSpec((XB_W, blk), lambda i: (0, i)),
        out_shape=jax.ShapeDtypeStruct((XB_W, N), jnp.float32),
    )(x8, d8)


@functools.cache
def _mesh():
    return plsc.VectorSubcoreMesh(core_axis_name="c", subcore_axis_name="s")


_SC_PARAMS = pltpu.CompilerParams(use_tc_tiling_on_sc=False,
                                  needs_layout_passes=False)


def _wid():
    return lax.axis_index("c") * 16 + lax.axis_index("s")


IOTA16 = lambda: lax.iota(jnp.int32, 16)


def _rank_last(m16, scr16):
    iota = IOTA16()
    sk, sl = plsc.sort_key_val(m16, iota)
    scr16[...] = sk
    prevk = plsc.load_gather(scr16, [jnp.maximum(iota - 1, 0)])
    nextk = plsc.load_gather(scr16, [jnp.minimum(iota + 1, 15)])
    isb = (iota == 0) | (prevk != sk)
    start = plsc.cummax(jnp.where(isb, iota, 0))
    rank_sorted = iota - start
    last_sorted = (iota == 15) | (nextk != sk)
    plsc.store_scatter(scr16, [sl], rank_sorted)
    rank = scr16[...]
    plsc.store_scatter(scr16, [sl], last_sorted.astype(jnp.int32))
    last = scr16[...] != 0
    return rank, last


def _b1_body(xs_h, ys_h, zs_h, m_h, s1_h, tot_h, xv, yv, zv, mv, sv, hist,
             scr16):
    w = _wid()
    base = w * CH
    pltpu.sync_copy(xs_h.at[pl.ds(base, CH)], xv)
    pltpu.sync_copy(ys_h.at[pl.ds(base, CH)], yv)
    pltpu.sync_copy(zs_h.at[pl.ds(base, CH)], zv)

    def zero_body(t, _):
        hist[pl.ds(t * 16, 16)] = jnp.zeros((16,), jnp.int32)
        return 0
    lax.fori_loop(0, M // 16, zero_body, 0)

    def body(j, _):
        off = j * 16
        x16 = xv[pl.ds(off, 16)]
        y16 = yv[pl.ds(off, 16)]
        z16 = zv[pl.ds(off, 16)]
        def vox(v):
            return jnp.clip(v * float(GRID), 0.0, float(GRID - 1)).astype(jnp.int32)
        m16 = vox(x16) * (GRID * GRID) + vox(y16) * GRID + vox(z16)
        mv[pl.ds(off, 16)] = m16
        rank, lastm = _rank_last(m16, scr16)
        old = plsc.load_gather(hist, [m16])
        sv[pl.ds(off, 16)] = old + rank
        plsc.store_scatter(hist, [m16], old + rank + 1, mask=lastm)
        return 0
    lax.fori_loop(0, CH // 16, body, 0)

    pltpu.sync_copy(mv, m_h.at[w])
    pltpu.sync_copy(sv, s1_h.at[w])
    pltpu.sync_copy(hist, tot_h.at[w])


def _b1(xs, ys, zs):
    return pl.kernel(
        _b1_body,
        out_type=[
            jax.ShapeDtypeStruct((NW, CH), jnp.int32),
            jax.ShapeDtypeStruct((NW, CH), jnp.int32),
            jax.ShapeDtypeStruct((NW, M), jnp.int32),
        ],
        mesh=_mesh(),
        compiler_params=_SC_PARAMS,
        scratch_types=[
            pltpu.VMEM((CH,), jnp.float32),
            pltpu.VMEM((CH,), jnp.float32),
            pltpu.VMEM((CH,), jnp.float32),
            pltpu.VMEM((CH,), jnp.int32),
            pltpu.VMEM((CH,), jnp.int32),
            pltpu.VMEM((M,), jnp.int32),
            pltpu.VMEM((16,), jnp.int32),
        ],
    )(xs, ys, zs)


def _b15_body(tot_h, pre_h, buf, prebuf, sem):
    u = _wid()
    mpw = M // NW
    copies = [pltpu.make_async_copy(
        tot_h.at[w2, pl.ds(u * mpw, mpw)], buf.at[w2], sem)
        for w2 in range(NW)]
    for cp in copies:
        cp.start()
    for cp in copies:
        cp.wait()

    def g_body(g, _):
        def w_body(w2, run):
            prebuf[w2, pl.ds(g * 16, 16)] = run
            return run + plsc.load_gather(
                buf, [jnp.full((16,), w2, jnp.int32), IOTA16() + g * 16])
        lax.fori_loop(0, NW, w_body, jnp.zeros((16,), jnp.int32))
        return 0
    lax.fori_loop(0, mpw // 16, g_body, 0)

    pltpu.sync_copy(prebuf, pre_h.at[:, pl.ds(u * mpw, mpw)])


def _b15(tot):
    return pl.kernel(
        _b15_body,
        out_type=jax.ShapeDtypeStruct((NW, M), jnp.int32),
        mesh=_mesh(),
        compiler_params=_SC_PARAMS,
        scratch_types=[
            pltpu.VMEM((NW, M // NW), jnp.int32),
            pltpu.VMEM((NW, M // NW), jnp.int32),
            pltpu.SemaphoreType.DMA,
        ],
    )(tot)


def _b2_body(m_h, s1_h, pre_h, ep_h, xp_h, dest_h,
             acc, mv, sv, dv, epb, sem):
    w = _wid()
    pltpu.sync_copy(pre_h.at[w], acc)
    pltpu.sync_copy(m_h.at[w], mv)
    pltpu.sync_copy(s1_h.at[w], sv)

    def body(j, _):
        off = j * 16
        m16 = mv[pl.ds(off, 16)]
        slot = sv[pl.ds(off, 16)] + plsc.load_gather(acc, [m16])
        slot = jnp.minimum(slot, C - 1)
        dest16 = m16 * C + slot
        dv[j // 8, pl.ds((j % 8) * 16, 16)] = dest16
        return 0
    lax.fori_loop(0, CH // 16, body, 0)

    pltpu.sync_copy(dv, dest_h.at[w])
    base = w * CH
    pltpu.sync_copy(ep_h.at[pl.ds(base, CH), :], epb)
    copies = []
    for t in range(CH // 128):
        copies.append(pltpu.make_async_copy(
            epb.at[pl.ds(t * 128, 128), :], xp_h.at[dv.at[t]], sem))
    for cp in copies:
        cp.start()
    for cp in copies:
        cp.wait()


def _b2(m_all, s1, pre, ep):
    return pl.kernel(
        _b2_body,
        out_type=[
            jax.ShapeDtypeStruct((ROWS, XB_W), jnp.bfloat16),
            jax.ShapeDtypeStruct((NW, CH // 128, 128), jnp.int32),
        ],
        mesh=_mesh(),
        compiler_params=_SC_PARAMS,
        scratch_types=[
            pltpu.VMEM((M,), jnp.int32),
            pltpu.VMEM((CH,), jnp.int32),
            pltpu.VMEM((CH,), jnp.int32),
            pltpu.VMEM((CH // 128, 128), jnp.int32),
            pltpu.VMEM((CH, XB_W), jnp.bfloat16),
            pltpu.SemaphoreType.DMA,
        ],
    )(m_all, s1, pre, ep)


def _c_body(xb_ref, w0_ref, b0_ref, w1_ref, b1_ref, fw_ref, fb_ref,
            vw_ref, vb_ref, yw_ref, yb_ref, y_ref):
    f32 = jnp.float32
    bf16 = jnp.bfloat16
    dims_n = (((1,), (0,)), ((), ()))

    X = xb_ref[...]
    X64 = X[:, 0:EP_W]

    def rep(b_ref, width):
        b = b_ref[...]
        return jnp.broadcast_to(b[:, None, :], (MB, C, width)).reshape(MB * C, width)

    def layer_v(hval, w_ref2):
        wb = w_ref2[...]
        outs = [lax.dot_general(hval[i * C:(i + 1) * C, :], wb[i], dims_n,
                                preferred_element_type=f32)
                for i in range(MB)]
        return jnp.concatenate(outs, axis=0)

    H1 = jax.nn.relu(layer_v(X64, w0_ref) + rep(b0_ref, 32)).astype(bf16)
    H2 = jax.nn.relu(layer_v(H1, w1_ref) + rep(b1_ref, 32)).astype(bf16)
    FT = (layer_v(H2, fw_ref) + rep(fb_ref, 32)).astype(bf16)
    HVin = jnp.concatenate([FT, X[:, EP_W:XB_W]], axis=1)
    HV = jax.nn.relu(layer_v(HVin, vw_ref) + rep(vb_ref, 32)).astype(bf16)
    YC = jnp.concatenate([HV, H2], axis=1)
    Y = layer_v(YC, yw_ref) + rep(yb_ref, Y_W)
    y_ref[...] = Y


def _c(xb, w0, b0, w1, b1, fw, fb, vw, vb, yw, yb):
    nsteps = M // MB
    mspec = lambda shp: pl.BlockSpec((MB,) + shp, lambda i: (i,) + (0,) * len(shp))
    return pl.pallas_call(
        _c_body,
        grid=(nsteps,),
        in_specs=[
            pl.BlockSpec((MB * C, XB_W), lambda i: (i, 0)),
            mspec((64, 32)), mspec((32,)),
            mspec((32, 32)), mspec((32,)),
            mspec((32, 32)), mspec((32,)),
            mspec((64, 32)), mspec((32,)),
            mspec((64, Y_W)), mspec((Y_W,)),
        ],
        out_specs=pl.BlockSpec((MB * C, Y_W), lambda i: (i, 0)),
        out_shape=jax.ShapeDtypeStruct((ROWS, Y_W), jnp.float32),
    )(xb, w0, b0, w1, b1, fw, fb, vw, vb, yw, yb)


def _d_body(y_h, dest_h, out_h, dv, yb, sem):
    w = _wid()
    pltpu.sync_copy(dest_h.at[w], dv)
    for t in range(CH // 128):
        cp = pltpu.make_async_copy(y_h.at[dv.at[t]], yb, sem)
        cp.start()
        cp.wait()
        pltpu.sync_copy(yb, out_h.at[pl.ds(w * CH + t * 128, 128), :])


def _d(y, dest):
    return pl.kernel(
        _d_body,
        out_type=jax.ShapeDtypeStruct((N, Y_W), jnp.float32),
        mesh=_mesh(),
        compiler_params=_SC_PARAMS,
        scratch_types=[
            pltpu.VMEM((CH // 128, 128), jnp.int32),
            pltpu.VMEM((128, Y_W), jnp.float32),
            pltpu.SemaphoreType.DMA,
        ],
    )(y, dest)


def kernel(pts, viewdirs, pts_w0, pts_b0, pts_w1, pts_b1, feat_w, feat_b,
           sigma_w, sigma_b, view_w, view_b, rgb_w, rgb_b):
    bf16 = jnp.bfloat16
    pts_flat = pts.reshape(N, 3)
    x8 = jnp.pad(pts_flat.T, ((0, 5), (0, 0)))
    d8 = jnp.pad(viewdirs.T, ((0, 5), (0, 0)))

    eT = _encode(x8, d8)
    ep = eT.T.astype(bf16)

    m_all, s1, tot = _b1(x8[0], x8[1], x8[2])
    pre = _b15(tot)
    xb, dest = _b2(m_all, s1, pre, ep)

    def tpad(wm, rows):
        wt = jnp.transpose(wm.astype(bf16), (0, 2, 1))
        return jnp.pad(wt, ((0, 0), (0, rows - wt.shape[1]), (0, 0)))
    w0T = tpad(pts_w0, 64)
    w1T = tpad(pts_w1, 32)
    fwT = tpad(feat_w, 32)
    vwT = tpad(view_w, 64)

    rgb_wT = jnp.transpose(rgb_w.astype(bf16), (0, 2, 1))
    top = jnp.pad(rgb_wT, ((0, 0), (0, 0), (0, Y_W - 3)))
    bot = jnp.pad(sigma_w.astype(bf16).reshape(M, 32, 1),
                  ((0, 0), (0, 0), (3, Y_W - 4)))
    yw = jnp.concatenate([top, bot], axis=1)
    yb = (jnp.pad(rgb_b, ((0, 0), (0, Y_W - 3)))
          + jnp.pad(sigma_b, ((0, 0), (3, Y_W - 4))))

    y = _c(xb, w0T, pts_b0, w1T, pts_b1, fwT, feat_b, vwT, view_b, yw, yb)
    res = _d(y, dest)

    rgb = res[:, 0:3].reshape(NRAYS, NSAMP, 3)
    sigma = res[:, 3:4].reshape(NRAYS, NSAMP, 1)
    return (rgb, sigma)


# speedup vs baseline: 1.3463x; 1.0894x over previous
import functools

import jax
import jax.numpy as jnp
from jax import lax
from jax.experimental import pallas as pl
from jax.experimental.pallas import tpu as pltpu
from jax.experimental.pallas import tpu_sc as plsc

GRID = 16
M = GRID * GRID * GRID
N = 32768
NRAYS = 1024
NSAMP = 32
C = 32
ROWS = M * C
L_XYZ = 10
L_DIR = 4
NW = 32
CH = N // NW
MB = 32
EP_W = 64
ED_W = 32
XB_W = EP_W + ED_W
Y_W = 16


def _sincos_rows(p, nlev):
    s = jnp.sin(p)
    c = jnp.cos(p)
    pieces = [p[0:3, :]]
    for i in range(nlev):
        pieces.append(s[0:3, :])
        pieces.append(c[0:3, :])
        if i + 1 < nlev:
            s, c = 2.0 * s * c, 1.0 - 2.0 * s * s
    return pieces


def _encode_body(x_ref, d_ref, o_ref):
    blk = x_ref.shape[1]
    rblk = d_ref.shape[1]
    pieces = _sincos_rows(x_ref[...], L_XYZ)
    off = 0
    for pc in pieces:
        o_ref[pl.ds(off, 3), :] = pc
        off += 3
    o_ref[pl.ds(off, EP_W - off), :] = jnp.zeros((EP_W - off, blk), jnp.float32)
    dpieces = _sincos_rows(d_ref[...], L_DIR)
    dpieces.append(jnp.zeros((ED_W - 3 * (1 + 2 * L_DIR), rblk), jnp.float32))
    dire = jnp.concatenate(dpieces, axis=0)
    diree = jnp.broadcast_to(dire[:, :, None], (ED_W, rblk, NSAMP))
    o_ref[pl.ds(EP_W, ED_W), :] = diree.reshape(ED_W, blk)


def _encode(x8, d8):
    blk = 4096
    rblk = blk // NSAMP
    return pl.pallas_call(
        _encode_body,
        grid=(N // blk,),
        in_specs=[pl.BlockSpec((8, blk), lambda i: (0, i)),
                  pl.BlockSpec((8, rblk), lambda i: (0, i))],
        out_specs=pl.BlockSpec((XB_W, blk), lambda i: (0, i)),
        out_shape=jax.ShapeDtypeStruct((XB_W, N), jnp.float32),
    )(x8, d8)


@functools.cache
def _mesh():
    return plsc.VectorSubcoreMesh(core_axis_name="c", subcore_axis_name="s")


_SC_PARAMS = pltpu.CompilerParams(use_tc_tiling_on_sc=False,
                                  needs_layout_passes=False)


def _wid():
    return lax.axis_index("c") * 16 + lax.axis_index("s")


IOTA16 = lambda: lax.iota(jnp.int32, 16)


def _rank_last(m16, scr16):
    iota = IOTA16()
    sk, sl = plsc.sort_key_val(m16, iota)
    scr16[...] = sk
    prevk = plsc.load_gather(scr16, [jnp.maximum(iota - 1, 0)])
    nextk = plsc.load_gather(scr16, [jnp.minimum(iota + 1, 15)])
    isb = (iota == 0) | (prevk != sk)
    start = plsc.cummax(jnp.where(isb, iota, 0))
    rank_sorted = iota - start
    last_sorted = (iota == 15) | (nextk != sk)
    plsc.store_scatter(scr16, [sl], rank_sorted)
    rank = scr16[...]
    plsc.store_scatter(scr16, [sl], last_sorted.astype(jnp.int32))
    last = scr16[...] != 0
    return rank, last


def _b1_body(xs_h, ys_h, zs_h, m_h, s1_h, tot_h, xv, yv, zv, mv, sv, hist,
             scr16):
    w = _wid()
    base = w * CH
    pltpu.sync_copy(xs_h.at[pl.ds(base, CH)], xv)
    pltpu.sync_copy(ys_h.at[pl.ds(base, CH)], yv)
    pltpu.sync_copy(zs_h.at[pl.ds(base, CH)], zv)

    def zero_body(t, _):
        hist[pl.ds(t * 16, 16)] = jnp.zeros((16,), jnp.int32)
        return 0
    lax.fori_loop(0, M // 16, zero_body, 0)

    def body(j, _):
        off = j * 16
        x16 = xv[pl.ds(off, 16)]
        y16 = yv[pl.ds(off, 16)]
        z16 = zv[pl.ds(off, 16)]
        def vox(v):
            return jnp.clip(v * float(GRID), 0.0, float(GRID - 1)).astype(jnp.int32)
        m16 = vox(x16) * (GRID * GRID) + vox(y16) * GRID + vox(z16)
        mv[pl.ds(off, 16)] = m16
        rank, lastm = _rank_last(m16, scr16)
        old = plsc.load_gather(hist, [m16])
        sv[pl.ds(off, 16)] = old + rank
        plsc.store_scatter(hist, [m16], old + rank + 1, mask=lastm)
        return 0
    lax.fori_loop(0, CH // 16, body, 0)

    pltpu.sync_copy(mv, m_h.at[w])
    pltpu.sync_copy(sv, s1_h.at[w])
    pltpu.sync_copy(hist, tot_h.at[w])


def _b1(xs, ys, zs):
    return pl.kernel(
        _b1_body,
        out_type=[
            jax.ShapeDtypeStruct((NW, CH), jnp.int32),
            jax.ShapeDtypeStruct((NW, CH), jnp.int32),
            jax.ShapeDtypeStruct((NW, M), jnp.int32),
        ],
        mesh=_mesh(),
        compiler_params=_SC_PARAMS,
        scratch_types=[
            pltpu.VMEM((CH,), jnp.float32),
            pltpu.VMEM((CH,), jnp.float32),
            pltpu.VMEM((CH,), jnp.float32),
            pltpu.VMEM((CH,), jnp.int32),
            pltpu.VMEM((CH,), jnp.int32),
            pltpu.VMEM((M,), jnp.int32),
            pltpu.VMEM((16,), jnp.int32),
        ],
    )(xs, ys, zs)


def _b15_body(tot_h, pre_h, buf, prebuf, sem):
    u = _wid()
    mpw = M // NW
    copies = [pltpu.make_async_copy(
        tot_h.at[w2, pl.ds(u * mpw, mpw)], buf.at[w2], sem)
        for w2 in range(NW)]
    for cp in copies:
        cp.start()
    for cp in copies:
        cp.wait()

    def g_body(g, _):
        def w_body(w2, run):
            prebuf[w2, pl.ds(g * 16, 16)] = run
            return run + plsc.load_gather(
                buf, [jnp.full((16,), w2, jnp.int32), IOTA16() + g * 16])
        lax.fori_loop(0, NW, w_body, jnp.zeros((16,), jnp.int32))
        return 0
    lax.fori_loop(0, mpw // 16, g_body, 0)

    pltpu.sync_copy(prebuf, pre_h.at[:, pl.ds(u * mpw, mpw)])


def _b15(tot):
    return pl.kernel(
        _b15_body,
        out_type=jax.ShapeDtypeStruct((NW, M), jnp.int32),
        mesh=_mesh(),
        compiler_params=_SC_PARAMS,
        scratch_types=[
            pltpu.VMEM((NW, M // NW), jnp.int32),
            pltpu.VMEM((NW, M // NW), jnp.int32),
            pltpu.SemaphoreType.DMA,
        ],
    )(tot)


def _b2_body(m_h, s1_h, pre_h, ep_h, xp_h, dest_h,
             acc, mv, sv, dv, epb, sem):
    w = _wid()
    pltpu.sync_copy(pre_h.at[w], acc)
    pltpu.sync_copy(m_h.at[w], mv)
    pltpu.sync_copy(s1_h.at[w], sv)

    def body(j, _):
        off = j * 16
        m16 = mv[pl.ds(off, 16)]
        slot = sv[pl.ds(off, 16)] + plsc.load_gather(acc, [m16])
        slot = jnp.minimum(slot, C - 1)
        dest16 = m16 * C + slot
        dv[j // 8, pl.ds((j % 8) * 16, 16)] = dest16
        return 0
    lax.fori_loop(0, CH // 16, body, 0)

    pltpu.sync_copy(dv, dest_h.at[w])
    base = w * CH
    pltpu.sync_copy(ep_h.at[pl.ds(base, CH), :], epb)
    copies = []
    for t in range(CH // 128):
        copies.append(pltpu.make_async_copy(
            epb.at[pl.ds(t * 128, 128), :], xp_h.at[dv.at[t]], sem))
    for cp in copies:
        cp.start()
    for cp in copies:
        cp.wait()


def _b2(m_all, s1, pre, ep):
    return pl.kernel(
        _b2_body,
        out_type=[
            jax.ShapeDtypeStruct((ROWS, XB_W), jnp.bfloat16),
            jax.ShapeDtypeStruct((NW, CH // 128, 128), jnp.int32),
        ],
        mesh=_mesh(),
        compiler_params=_SC_PARAMS,
        scratch_types=[
            pltpu.VMEM((M,), jnp.int32),
            pltpu.VMEM((CH,), jnp.int32),
            pltpu.VMEM((CH,), jnp.int32),
            pltpu.VMEM((CH // 128, 128), jnp.int32),
            pltpu.VMEM((CH, XB_W), jnp.bfloat16),
            pltpu.SemaphoreType.DMA,
        ],
    )(m_all, s1, pre, ep)


def _c_body(xb_ref, w0_ref, b0_ref, w1_ref, b1_ref, fw_ref, fb_ref,
            vw_ref, vb_ref, yw_ref, yb_ref, y_ref):
    f32 = jnp.float32
    bf16 = jnp.bfloat16
    dims_n = (((1,), (0,)), ((), ()))

    X = xb_ref[...]
    X64 = X[:, 0:EP_W]

    def rep(b_ref, width):
        b = b_ref[...]
        return jnp.broadcast_to(b[:, None, :], (MB, C, width)).reshape(MB * C, width)

    def layer_v(hval, w_ref2):
        wb = w_ref2[...]
        outs = [lax.dot_general(hval[i * C:(i + 1) * C, :], wb[i], dims_n,
                                preferred_element_type=f32)
                for i in range(MB)]
        return jnp.concatenate(outs, axis=0)

    H1 = jax.nn.relu(layer_v(X64, w0_ref) + rep(b0_ref, 32)).astype(bf16)
    H2 = jax.nn.relu(layer_v(H1, w1_ref) + rep(b1_ref, 32)).astype(bf16)
    FT = (layer_v(H2, fw_ref) + rep(fb_ref, 32)).astype(bf16)
    HVin = jnp.concatenate([FT, X[:, EP_W:XB_W]], axis=1)
    HV = jax.nn.relu(layer_v(HVin, vw_ref) + rep(vb_ref, 32)).astype(bf16)
    YC = jnp.concatenate([HV, H2], axis=1)
    Y = layer_v(YC, yw_ref) + rep(yb_ref, Y_W)
    y_ref[...] = Y


def _c(xb, w0, b0, w1, b1, fw, fb, vw, vb, yw, yb):
    nsteps = M // MB
    mspec = lambda shp: pl.BlockSpec((MB,) + shp, lambda i: (i,) + (0,) * len(shp))
    return pl.pallas_call(
        _c_body,
        grid=(nsteps,),
        in_specs=[
            pl.BlockSpec((MB * C, XB_W), lambda i: (i, 0)),
            mspec((64, 32)), mspec((32,)),
            mspec((32, 32)), mspec((32,)),
            mspec((32, 32)), mspec((32,)),
            mspec((64, 32)), mspec((32,)),
            mspec((64, Y_W)), mspec((Y_W,)),
        ],
        out_specs=pl.BlockSpec((MB * C, Y_W), lambda i: (i, 0)),
        out_shape=jax.ShapeDtypeStruct((ROWS, Y_W), jnp.float32),
    )(xb, w0, b0, w1, b1, fw, fb, vw, vb, yw, yb)


def _d_body(y_h, dest_h, out_h, dv, yb, sem):
    w = _wid()
    pltpu.sync_copy(dest_h.at[w], dv)
    for t in range(CH // 128):
        cp = pltpu.make_async_copy(y_h.at[dv.at[t]], yb, sem)
        cp.start()
        cp.wait()
        pltpu.sync_copy(yb, out_h.at[pl.ds(w * CH + t * 128, 128), :])


def _d(y, dest):
    return pl.kernel(
        _d_body,
        out_type=jax.ShapeDtypeStruct((N, Y_W), jnp.float32),
        mesh=_mesh(),
        compiler_params=_SC_PARAMS,
        scratch_types=[
            pltpu.VMEM((CH // 128, 128), jnp.int32),
            pltpu.VMEM((128, Y_W), jnp.float32),
            pltpu.SemaphoreType.DMA,
        ],
    )(y, dest)


def kernel(pts, viewdirs, pts_w0, pts_b0, pts_w1, pts_b1, feat_w, feat_b,
           sigma_w, sigma_b, view_w, view_b, rgb_w, rgb_b):
    bf16 = jnp.bfloat16
    pts_flat = pts.reshape(N, 3)
    x8 = jnp.pad(pts_flat.T, ((0, 5), (0, 0)))
    d8 = jnp.pad(viewdirs.T, ((0, 5), (0, 0)))

    eT = _encode(x8, d8)
    ep = eT.T.astype(bf16)

    m_all, s1, tot = _b1(x8[0], x8[1], x8[2])
    pre = _b15(tot)
    xb, dest = _b2(m_all, s1, pre, ep)

    def tpad(wm, rows):
        wt = jnp.transpose(wm.astype(bf16), (0, 2, 1))
        return jnp.pad(wt, ((0, 0), (0, rows - wt.shape[1]), (0, 0)))
    w0T = tpad(pts_w0, 64)
    w1T = tpad(pts_w1, 32)
    fwT = tpad(feat_w, 32)
    vwT = tpad(view_w, 64)

    rgb_wT = jnp.transpose(rgb_w.astype(bf16), (0, 2, 1))
    top = jnp.pad(rgb_wT, ((0, 0), (0, 0), (0, Y_W - 3)))
    bot = jnp.pad(sigma_w.astype(bf16).reshape(M, 32, 1),
                  ((0, 0), (0, 0), (3, Y_W - 4)))
    yw = jnp.concatenate([top, bot], axis=1)
    yb = (jnp.pad(rgb_b, ((0, 0), (0, Y_W - 3)))
          + jnp.pad(sigma_b, ((0, 0), (3, Y_W - 4))))

    y = _c(xb, w0T, pts_b0, w1T, pts_b1, fwT, feat_b, vwT, view_b, yw, yb)
    res = _d(y, dest)

    rgb = res[:, 0:3].reshape(NRAYS, NSAMP, 3)
    sigma = res[:, 3:4].reshape(NRAYS, NSAMP, 1)
    return (rgb, sigma)


# speedup vs baseline: 1.4291x; 1.0615x over previous
import functools

import jax
import jax.numpy as jnp
from jax import lax
from jax.experimental import pallas as pl
from jax.experimental.pallas import tpu as pltpu
from jax.experimental.pallas import tpu_sc as plsc

GRID = 16
M = GRID * GRID * GRID
N = 32768
NRAYS = 1024
NSAMP = 32
C = 32
ROWS = M * C
L_XYZ = 10
L_DIR = 4
NW = 32
CH = N // NW
MB = 64
EP_W = 64
ED_W = 32
XB_W = EP_W + ED_W
Y_W = 16


def _sincos_rows(p, nlev):
    s = jnp.sin(p)
    c = jnp.cos(p)
    pieces = [p[0:3, :]]
    for i in range(nlev):
        pieces.append(s[0:3, :])
        pieces.append(c[0:3, :])
        if i + 1 < nlev:
            s, c = 2.0 * s * c, 1.0 - 2.0 * s * s
    return pieces


def _encode_body(x_ref, d_ref, o_ref):
    blk = x_ref.shape[1]
    rblk = d_ref.shape[1]
    pieces = _sincos_rows(x_ref[...], L_XYZ)
    off = 0
    for pc in pieces:
        o_ref[pl.ds(off, 3), :] = pc
        off += 3
    o_ref[pl.ds(off, EP_W - off), :] = jnp.zeros((EP_W - off, blk), jnp.float32)
    dpieces = _sincos_rows(d_ref[...], L_DIR)
    dpieces.append(jnp.zeros((ED_W - 3 * (1 + 2 * L_DIR), rblk), jnp.float32))
    dire = jnp.concatenate(dpieces, axis=0)
    diree = jnp.broadcast_to(dire[:, :, None], (ED_W, rblk, NSAMP))
    o_ref[pl.ds(EP_W, ED_W), :] = diree.reshape(ED_W, blk)


def _encode(x8, d8):
    blk = 4096
    rblk = blk // NSAMP
    return pl.pallas_call(
        _encode_body,
        grid=(N // blk,),
        in_specs=[pl.BlockSpec((8, blk), lambda i: (0, i)),
                  pl.BlockSpec((8, rblk), lambda i: (0, i))],
        out_specs=pl.BlockSpec((XB_W, blk), lambda i: (0, i)),
        out_shape=jax.ShapeDtypeStruct((XB_W, N), jnp.float32),
    )(x8, d8)


@functools.cache
def _mesh():
    return plsc.VectorSubcoreMesh(core_axis_name="c", subcore_axis_name="s")


_SC_PARAMS = pltpu.CompilerParams(use_tc_tiling_on_sc=False,
                                  needs_layout_passes=False)


def _wid():
    return lax.axis_index("c") * 16 + lax.axis_index("s")


IOTA16 = lambda: lax.iota(jnp.int32, 16)


def _rank_last(m16, scr16):
    iota = IOTA16()
    sk, sl = plsc.sort_key_val(m16, iota)
    scr16[...] = sk
    prevk = plsc.load_gather(scr16, [jnp.maximum(iota - 1, 0)])
    nextk = plsc.load_gather(scr16, [jnp.minimum(iota + 1, 15)])
    isb = (iota == 0) | (prevk != sk)
    start = plsc.cummax(jnp.where(isb, iota, 0))
    rank_sorted = iota - start
    last_sorted = (iota == 15) | (nextk != sk)
    plsc.store_scatter(scr16, [sl], rank_sorted)
    rank = scr16[...]
    plsc.store_scatter(scr16, [sl], last_sorted.astype(jnp.int32))
    last = scr16[...] != 0
    return rank, last


def _b1_body(xs_h, ys_h, zs_h, m_h, s1_h, tot_h, xv, yv, zv, mv, sv, hist,
             scr16):
    w = _wid()
    base = w * CH
    pltpu.sync_copy(xs_h.at[pl.ds(base, CH)], xv)
    pltpu.sync_copy(ys_h.at[pl.ds(base, CH)], yv)
    pltpu.sync_copy(zs_h.at[pl.ds(base, CH)], zv)

    def zero_body(t, _):
        hist[pl.ds(t * 16, 16)] = jnp.zeros((16,), jnp.int32)
        return 0
    lax.fori_loop(0, M // 16, zero_body, 0)

    def body(j, _):
        off = j * 16
        x16 = xv[pl.ds(off, 16)]
        y16 = yv[pl.ds(off, 16)]
        z16 = zv[pl.ds(off, 16)]
        def vox(v):
            return jnp.clip(v * float(GRID), 0.0, float(GRID - 1)).astype(jnp.int32)
        m16 = vox(x16) * (GRID * GRID) + vox(y16) * GRID + vox(z16)
        mv[pl.ds(off, 16)] = m16
        rank, lastm = _rank_last(m16, scr16)
        old = plsc.load_gather(hist, [m16])
        sv[pl.ds(off, 16)] = old + rank
        plsc.store_scatter(hist, [m16], old + rank + 1, mask=lastm)
        return 0
    lax.fori_loop(0, CH // 16, body, 0)

    pltpu.sync_copy(mv, m_h.at[w])
    pltpu.sync_copy(sv, s1_h.at[w])
    pltpu.sync_copy(hist, tot_h.at[w])


def _b1(xs, ys, zs):
    return pl.kernel(
        _b1_body,
        out_type=[
            jax.ShapeDtypeStruct((NW, CH), jnp.int32),
            jax.ShapeDtypeStruct((NW, CH), jnp.int32),
            jax.ShapeDtypeStruct((NW, M), jnp.int32),
        ],
        mesh=_mesh(),
        compiler_params=_SC_PARAMS,
        scratch_types=[
            pltpu.VMEM((CH,), jnp.float32),
            pltpu.VMEM((CH,), jnp.float32),
            pltpu.VMEM((CH,), jnp.float32),
            pltpu.VMEM((CH,), jnp.int32),
            pltpu.VMEM((CH,), jnp.int32),
            pltpu.VMEM((M,), jnp.int32),
            pltpu.VMEM((16,), jnp.int32),
        ],
    )(xs, ys, zs)


def _b15_body(tot_h, pre_h, buf, prebuf, sem):
    u = _wid()
    mpw = M // NW
    copies = [pltpu.make_async_copy(
        tot_h.at[w2, pl.ds(u * mpw, mpw)], buf.at[w2], sem)
        for w2 in range(NW)]
    for cp in copies:
        cp.start()
    for cp in copies:
        cp.wait()

    def g_body(g, _):
        def w_body(w2, run):
            prebuf[w2, pl.ds(g * 16, 16)] = run
            return run + plsc.load_gather(
                buf, [jnp.full((16,), w2, jnp.int32), IOTA16() + g * 16])
        lax.fori_loop(0, NW, w_body, jnp.zeros((16,), jnp.int32))
        return 0
    lax.fori_loop(0, mpw // 16, g_body, 0)

    pltpu.sync_copy(prebuf, pre_h.at[:, pl.ds(u * mpw, mpw)])


def _b15(tot):
    return pl.kernel(
        _b15_body,
        out_type=jax.ShapeDtypeStruct((NW, M), jnp.int32),
        mesh=_mesh(),
        compiler_params=_SC_PARAMS,
        scratch_types=[
            pltpu.VMEM((NW, M // NW), jnp.int32),
            pltpu.VMEM((NW, M // NW), jnp.int32),
            pltpu.SemaphoreType.DMA,
        ],
    )(tot)


def _b2_body(m_h, s1_h, pre_h, ep_h, xp_h, dest_h,
             acc, mv, sv, dv, epb, sem):
    w = _wid()
    pltpu.sync_copy(pre_h.at[w], acc)
    pltpu.sync_copy(m_h.at[w], mv)
    pltpu.sync_copy(s1_h.at[w], sv)

    def body(j, _):
        off = j * 16
        m16 = mv[pl.ds(off, 16)]
        slot = sv[pl.ds(off, 16)] + plsc.load_gather(acc, [m16])
        slot = jnp.minimum(slot, C - 1)
        dest16 = m16 * C + slot
        dv[j // 8, pl.ds((j % 8) * 16, 16)] = dest16
        return 0
    lax.fori_loop(0, CH // 16, body, 0)

    pltpu.sync_copy(dv, dest_h.at[w])
    base = w * CH
    pltpu.sync_copy(ep_h.at[pl.ds(base, CH), :], epb)
    copies = []
    for t in range(CH // 128):
        copies.append(pltpu.make_async_copy(
            epb.at[pl.ds(t * 128, 128), :], xp_h.at[dv.at[t]], sem))
    for cp in copies:
        cp.start()
    for cp in copies:
        cp.wait()


def _b2(m_all, s1, pre, ep):
    return pl.kernel(
        _b2_body,
        out_type=[
            jax.ShapeDtypeStruct((ROWS, XB_W), jnp.bfloat16),
            jax.ShapeDtypeStruct((NW, CH // 128, 128), jnp.int32),
        ],
        mesh=_mesh(),
        compiler_params=_SC_PARAMS,
        scratch_types=[
            pltpu.VMEM((M,), jnp.int32),
            pltpu.VMEM((CH,), jnp.int32),
            pltpu.VMEM((CH,), jnp.int32),
            pltpu.VMEM((CH // 128, 128), jnp.int32),
            pltpu.VMEM((CH, XB_W), jnp.bfloat16),
            pltpu.SemaphoreType.DMA,
        ],
    )(m_all, s1, pre, ep)


def _c_body(xb_ref, w0_ref, b0_ref, w1_ref, b1_ref, fw_ref, fb_ref,
            vw_ref, vb_ref, yw_ref, yb_ref, y_ref):
    f32 = jnp.float32
    bf16 = jnp.bfloat16
    dims_n = (((1,), (0,)), ((), ()))

    X = xb_ref[...]
    X64 = X[:, 0:EP_W]

    def rep(b_ref, width):
        b = b_ref[...]
        return jnp.broadcast_to(b[:, None, :], (MB, C, width)).reshape(MB * C, width)

    def layer_v(hval, w_ref2):
        wb = w_ref2[...]
        outs = [lax.dot_general(hval[i * C:(i + 1) * C, :], wb[i], dims_n,
                                preferred_element_type=f32)
                for i in range(MB)]
        return jnp.concatenate(outs, axis=0)

    H1 = jax.nn.relu(layer_v(X64, w0_ref) + rep(b0_ref, 32)).astype(bf16)
    H2 = jax.nn.relu(layer_v(H1, w1_ref) + rep(b1_ref, 32)).astype(bf16)
    FT = (layer_v(H2, fw_ref) + rep(fb_ref, 32)).astype(bf16)
    HVin = jnp.concatenate([FT, X[:, EP_W:XB_W]], axis=1)
    HV = jax.nn.relu(layer_v(HVin, vw_ref) + rep(vb_ref, 32)).astype(bf16)
    YC = jnp.concatenate([HV, H2], axis=1)
    Y = layer_v(YC, yw_ref) + rep(yb_ref, Y_W)
    y_ref[...] = Y


def _c(xb, w0, b0, w1, b1, fw, fb, vw, vb, yw, yb):
    nsteps = M // MB
    mspec = lambda shp: pl.BlockSpec((MB,) + shp, lambda i: (i,) + (0,) * len(shp))
    return pl.pallas_call(
        _c_body,
        grid=(nsteps,),
        in_specs=[
            pl.BlockSpec((MB * C, XB_W), lambda i: (i, 0)),
            mspec((64, 32)), mspec((32,)),
            mspec((32, 32)), mspec((32,)),
            mspec((32, 32)), mspec((32,)),
            mspec((64, 32)), mspec((32,)),
            mspec((64, Y_W)), mspec((Y_W,)),
        ],
        out_specs=pl.BlockSpec((MB * C, Y_W), lambda i: (i, 0)),
        out_shape=jax.ShapeDtypeStruct((ROWS, Y_W), jnp.float32),
    )(xb, w0, b0, w1, b1, fw, fb, vw, vb, yw, yb)


def _d_body(y_h, dest_h, out_h, dv, yb, sem):
    w = _wid()
    pltpu.sync_copy(dest_h.at[w], dv)
    for t in range(CH // 128):
        cp = pltpu.make_async_copy(y_h.at[dv.at[t]], yb, sem)
        cp.start()
        cp.wait()
        pltpu.sync_copy(yb, out_h.at[pl.ds(w * CH + t * 128, 128), :])


def _d(y, dest):
    return pl.kernel(
        _d_body,
        out_type=jax.ShapeDtypeStruct((N, Y_W), jnp.float32),
        mesh=_mesh(),
        compiler_params=_SC_PARAMS,
        scratch_types=[
            pltpu.VMEM((CH // 128, 128), jnp.int32),
            pltpu.VMEM((128, Y_W), jnp.float32),
            pltpu.SemaphoreType.DMA,
        ],
    )(y, dest)


def kernel(pts, viewdirs, pts_w0, pts_b0, pts_w1, pts_b1, feat_w, feat_b,
           sigma_w, sigma_b, view_w, view_b, rgb_w, rgb_b):
    bf16 = jnp.bfloat16
    pts_flat = pts.reshape(N, 3)
    x8 = jnp.pad(pts_flat.T, ((0, 5), (0, 0)))
    d8 = jnp.pad(viewdirs.T, ((0, 5), (0, 0)))

    eT = _encode(x8, d8)
    ep = eT.T.astype(bf16)

    m_all, s1, tot = _b1(x8[0], x8[1], x8[2])
    pre = _b15(tot)
    xb, dest = _b2(m_all, s1, pre, ep)

    def tpad(wm, rows):
        wt = jnp.transpose(wm.astype(bf16), (0, 2, 1))
        return jnp.pad(wt, ((0, 0), (0, rows - wt.shape[1]), (0, 0)))
    w0T = tpad(pts_w0, 64)
    w1T = tpad(pts_w1, 32)
    fwT = tpad(feat_w, 32)
    vwT = tpad(view_w, 64)

    rgb_wT = jnp.transpose(rgb_w.astype(bf16), (0, 2, 1))
    top = jnp.pad(rgb_wT, ((0, 0), (0, 0), (0, Y_W - 3)))
    bot = jnp.pad(sigma_w.astype(bf16).reshape(M, 32, 1),
                  ((0, 0), (0, 0), (3, Y_W - 4)))
    yw = jnp.concatenate([top, bot], axis=1)
    yb = (jnp.pad(rgb_b, ((0, 0), (0, Y_W - 3)))
          + jnp.pad(sigma_b, ((0, 0), (3, Y_W - 4))))

    y = _c(xb, w0T, pts_b0, w1T, pts_b1, fwT, feat_b, vwT, view_b, yw, yb)
    res = _d(y, dest)

    rgb = res[:, 0:3].reshape(NRAYS, NSAMP, 3)
    sigma = res[:, 3:4].reshape(NRAYS, NSAMP, 1)
    return (rgb, sigma)


# speedup vs baseline: 1.4628x; 1.0236x over previous
import functools

import jax
import jax.numpy as jnp
from jax import lax
from jax.experimental import pallas as pl
from jax.experimental.pallas import tpu as pltpu
from jax.experimental.pallas import tpu_sc as plsc

GRID = 16
M = GRID * GRID * GRID
N = 32768
NRAYS = 1024
NSAMP = 32
C = 32
ROWS = M * C
L_XYZ = 10
L_DIR = 4
NW = 32
CH = N // NW
MB = 128
EP_W = 64
ED_W = 32
XB_W = EP_W + ED_W
Y_W = 16


def _sincos_rows(p, nlev):
    s = jnp.sin(p)
    c = jnp.cos(p)
    pieces = [p[0:3, :]]
    for i in range(nlev):
        pieces.append(s[0:3, :])
        pieces.append(c[0:3, :])
        if i + 1 < nlev:
            s, c = 2.0 * s * c, 1.0 - 2.0 * s * s
    return pieces


def _encode_body(x_ref, d_ref, o_ref):
    blk = x_ref.shape[1]
    rblk = d_ref.shape[1]
    pieces = _sincos_rows(x_ref[...], L_XYZ)
    off = 0
    for pc in pieces:
        o_ref[pl.ds(off, 3), :] = pc
        off += 3
    o_ref[pl.ds(off, EP_W - off), :] = jnp.zeros((EP_W - off, blk), jnp.float32)
    dpieces = _sincos_rows(d_ref[...], L_DIR)
    dpieces.append(jnp.zeros((ED_W - 3 * (1 + 2 * L_DIR), rblk), jnp.float32))
    dire = jnp.concatenate(dpieces, axis=0)
    diree = jnp.broadcast_to(dire[:, :, None], (ED_W, rblk, NSAMP))
    o_ref[pl.ds(EP_W, ED_W), :] = diree.reshape(ED_W, blk)


def _encode(x8, d8):
    blk = 4096
    rblk = blk // NSAMP
    return pl.pallas_call(
        _encode_body,
        grid=(N // blk,),
        in_specs=[pl.BlockSpec((8, blk), lambda i: (0, i)),
                  pl.BlockSpec((8, rblk), lambda i: (0, i))],
        out_specs=pl.BlockSpec((XB_W, blk), lambda i: (0, i)),
        out_shape=jax.ShapeDtypeStruct((XB_W, N), jnp.float32),
    )(x8, d8)


@functools.cache
def _mesh():
    return plsc.VectorSubcoreMesh(core_axis_name="c", subcore_axis_name="s")


_SC_PARAMS = pltpu.CompilerParams(use_tc_tiling_on_sc=False,
                                  needs_layout_passes=False)


def _wid():
    return lax.axis_index("c") * 16 + lax.axis_index("s")


IOTA16 = lambda: lax.iota(jnp.int32, 16)


def _rank_last(m16, scr16):
    iota = IOTA16()
    sk, sl = plsc.sort_key_val(m16, iota)
    scr16[...] = sk
    prevk = plsc.load_gather(scr16, [jnp.maximum(iota - 1, 0)])
    nextk = plsc.load_gather(scr16, [jnp.minimum(iota + 1, 15)])
    isb = (iota == 0) | (prevk != sk)
    start = plsc.cummax(jnp.where(isb, iota, 0))
    rank_sorted = iota - start
    last_sorted = (iota == 15) | (nextk != sk)
    plsc.store_scatter(scr16, [sl], rank_sorted)
    rank = scr16[...]
    plsc.store_scatter(scr16, [sl], last_sorted.astype(jnp.int32))
    last = scr16[...] != 0
    return rank, last


def _b1_body(xs_h, ys_h, zs_h, m_h, s1_h, tot_h, xv, yv, zv, mv, sv, hist,
             scr16):
    w = _wid()
    base = w * CH
    pltpu.sync_copy(xs_h.at[pl.ds(base, CH)], xv)
    pltpu.sync_copy(ys_h.at[pl.ds(base, CH)], yv)
    pltpu.sync_copy(zs_h.at[pl.ds(base, CH)], zv)

    def zero_body(t, _):
        hist[pl.ds(t * 16, 16)] = jnp.zeros((16,), jnp.int32)
        return 0
    lax.fori_loop(0, M // 16, zero_body, 0)

    def body(j, _):
        off = j * 16
        x16 = xv[pl.ds(off, 16)]
        y16 = yv[pl.ds(off, 16)]
        z16 = zv[pl.ds(off, 16)]
        def vox(v):
            return jnp.clip(v * float(GRID), 0.0, float(GRID - 1)).astype(jnp.int32)
        m16 = vox(x16) * (GRID * GRID) + vox(y16) * GRID + vox(z16)
        mv[pl.ds(off, 16)] = m16
        rank, lastm = _rank_last(m16, scr16)
        old = plsc.load_gather(hist, [m16])
        sv[pl.ds(off, 16)] = old + rank
        plsc.store_scatter(hist, [m16], old + rank + 1, mask=lastm)
        return 0
    lax.fori_loop(0, CH // 16, body, 0)

    pltpu.sync_copy(mv, m_h.at[w])
    pltpu.sync_copy(sv, s1_h.at[w])
    pltpu.sync_copy(hist, tot_h.at[w])


def _b1(xs, ys, zs):
    return pl.kernel(
        _b1_body,
        out_type=[
            jax.ShapeDtypeStruct((NW, CH), jnp.int32),
            jax.ShapeDtypeStruct((NW, CH), jnp.int32),
            jax.ShapeDtypeStruct((NW, M), jnp.int32),
        ],
        mesh=_mesh(),
        compiler_params=_SC_PARAMS,
        scratch_types=[
            pltpu.VMEM((CH,), jnp.float32),
            pltpu.VMEM((CH,), jnp.float32),
            pltpu.VMEM((CH,), jnp.float32),
            pltpu.VMEM((CH,), jnp.int32),
            pltpu.VMEM((CH,), jnp.int32),
            pltpu.VMEM((M,), jnp.int32),
            pltpu.VMEM((16,), jnp.int32),
        ],
    )(xs, ys, zs)


def _b15_body(tot_h, pre_h, buf, prebuf, sem):
    u = _wid()
    mpw = M // NW
    copies = [pltpu.make_async_copy(
        tot_h.at[w2, pl.ds(u * mpw, mpw)], buf.at[w2], sem)
        for w2 in range(NW)]
    for cp in copies:
        cp.start()
    for cp in copies:
        cp.wait()

    def g_body(g, _):
        def w_body(w2, run):
            prebuf[w2, pl.ds(g * 16, 16)] = run
            return run + plsc.load_gather(
                buf, [jnp.full((16,), w2, jnp.int32), IOTA16() + g * 16])
        lax.fori_loop(0, NW, w_body, jnp.zeros((16,), jnp.int32))
        return 0
    lax.fori_loop(0, mpw // 16, g_body, 0)

    pltpu.sync_copy(prebuf, pre_h.at[:, pl.ds(u * mpw, mpw)])


def _b15(tot):
    return pl.kernel(
        _b15_body,
        out_type=jax.ShapeDtypeStruct((NW, M), jnp.int32),
        mesh=_mesh(),
        compiler_params=_SC_PARAMS,
        scratch_types=[
            pltpu.VMEM((NW, M // NW), jnp.int32),
            pltpu.VMEM((NW, M // NW), jnp.int32),
            pltpu.SemaphoreType.DMA,
        ],
    )(tot)


def _b2_body(m_h, s1_h, pre_h, ep_h, xp_h, dest_h,
             acc, mv, sv, dv, epb, sem):
    w = _wid()
    pltpu.sync_copy(pre_h.at[w], acc)
    pltpu.sync_copy(m_h.at[w], mv)
    pltpu.sync_copy(s1_h.at[w], sv)

    def body(j, _):
        off = j * 16
        m16 = mv[pl.ds(off, 16)]
        slot = sv[pl.ds(off, 16)] + plsc.load_gather(acc, [m16])
        slot = jnp.minimum(slot, C - 1)
        dest16 = m16 * C + slot
        dv[j // 8, pl.ds((j % 8) * 16, 16)] = dest16
        return 0
    lax.fori_loop(0, CH // 16, body, 0)

    pltpu.sync_copy(dv, dest_h.at[w])
    base = w * CH
    pltpu.sync_copy(ep_h.at[pl.ds(base, CH), :], epb)
    copies = []
    for t in range(CH // 128):
        copies.append(pltpu.make_async_copy(
            epb.at[pl.ds(t * 128, 128), :], xp_h.at[dv.at[t]], sem))
    for cp in copies:
        cp.start()
    for cp in copies:
        cp.wait()


def _b2(m_all, s1, pre, ep):
    return pl.kernel(
        _b2_body,
        out_type=[
            jax.ShapeDtypeStruct((ROWS, XB_W), jnp.bfloat16),
            jax.ShapeDtypeStruct((NW, CH // 128, 128), jnp.int32),
        ],
        mesh=_mesh(),
        compiler_params=_SC_PARAMS,
        scratch_types=[
            pltpu.VMEM((M,), jnp.int32),
            pltpu.VMEM((CH,), jnp.int32),
            pltpu.VMEM((CH,), jnp.int32),
            pltpu.VMEM((CH // 128, 128), jnp.int32),
            pltpu.VMEM((CH, XB_W), jnp.bfloat16),
            pltpu.SemaphoreType.DMA,
        ],
    )(m_all, s1, pre, ep)


def _c_body(xb_ref, w0_ref, b0_ref, w1_ref, b1_ref, fw_ref, fb_ref,
            vw_ref, vb_ref, yw_ref, yb_ref, y_ref):
    f32 = jnp.float32
    bf16 = jnp.bfloat16
    dims_n = (((1,), (0,)), ((), ()))

    X = xb_ref[...]
    X64 = X[:, 0:EP_W]

    def rep(b_ref, width):
        b = b_ref[...]
        return jnp.broadcast_to(b[:, None, :], (MB, C, width)).reshape(MB * C, width)

    def layer_v(hval, w_ref2):
        wb = w_ref2[...]
        outs = [lax.dot_general(hval[i * C:(i + 1) * C, :], wb[i], dims_n,
                                preferred_element_type=f32)
                for i in range(MB)]
        return jnp.concatenate(outs, axis=0)

    H1 = jax.nn.relu(layer_v(X64, w0_ref) + rep(b0_ref, 32)).astype(bf16)
    H2 = jax.nn.relu(layer_v(H1, w1_ref) + rep(b1_ref, 32)).astype(bf16)
    FT = (layer_v(H2, fw_ref) + rep(fb_ref, 32)).astype(bf16)
    HVin = jnp.concatenate([FT, X[:, EP_W:XB_W]], axis=1)
    HV = jax.nn.relu(layer_v(HVin, vw_ref) + rep(vb_ref, 32)).astype(bf16)
    YC = jnp.concatenate([HV, H2], axis=1)
    Y = layer_v(YC, yw_ref) + rep(yb_ref, Y_W)
    y_ref[...] = Y


def _c(xb, w0, b0, w1, b1, fw, fb, vw, vb, yw, yb):
    nsteps = M // MB
    mspec = lambda shp: pl.BlockSpec((MB,) + shp, lambda i: (i,) + (0,) * len(shp))
    return pl.pallas_call(
        _c_body,
        grid=(nsteps,),
        in_specs=[
            pl.BlockSpec((MB * C, XB_W), lambda i: (i, 0)),
            mspec((64, 32)), mspec((32,)),
            mspec((32, 32)), mspec((32,)),
            mspec((32, 32)), mspec((32,)),
            mspec((64, 32)), mspec((32,)),
            mspec((64, Y_W)), mspec((Y_W,)),
        ],
        out_specs=pl.BlockSpec((MB * C, Y_W), lambda i: (i, 0)),
        out_shape=jax.ShapeDtypeStruct((ROWS, Y_W), jnp.float32),
    )(xb, w0, b0, w1, b1, fw, fb, vw, vb, yw, yb)


def _d_body(y_h, dest_h, out_h, dv, yb, sem):
    w = _wid()
    pltpu.sync_copy(dest_h.at[w], dv)
    for t in range(CH // 128):
        cp = pltpu.make_async_copy(y_h.at[dv.at[t]], yb, sem)
        cp.start()
        cp.wait()
        pltpu.sync_copy(yb, out_h.at[pl.ds(w * CH + t * 128, 128), :])


def _d(y, dest):
    return pl.kernel(
        _d_body,
        out_type=jax.ShapeDtypeStruct((N, Y_W), jnp.float32),
        mesh=_mesh(),
        compiler_params=_SC_PARAMS,
        scratch_types=[
            pltpu.VMEM((CH // 128, 128), jnp.int32),
            pltpu.VMEM((128, Y_W), jnp.float32),
            pltpu.SemaphoreType.DMA,
        ],
    )(y, dest)


def kernel(pts, viewdirs, pts_w0, pts_b0, pts_w1, pts_b1, feat_w, feat_b,
           sigma_w, sigma_b, view_w, view_b, rgb_w, rgb_b):
    bf16 = jnp.bfloat16
    pts_flat = pts.reshape(N, 3)
    x8 = jnp.pad(pts_flat.T, ((0, 5), (0, 0)))
    d8 = jnp.pad(viewdirs.T, ((0, 5), (0, 0)))

    eT = _encode(x8, d8)
    ep = eT.T.astype(bf16)

    m_all, s1, tot = _b1(x8[0], x8[1], x8[2])
    pre = _b15(tot)
    xb, dest = _b2(m_all, s1, pre, ep)

    def tpad(wm, rows):
        wt = jnp.transpose(wm.astype(bf16), (0, 2, 1))
        return jnp.pad(wt, ((0, 0), (0, rows - wt.shape[1]), (0, 0)))
    w0T = tpad(pts_w0, 64)
    w1T = tpad(pts_w1, 32)
    fwT = tpad(feat_w, 32)
    vwT = tpad(view_w, 64)

    rgb_wT = jnp.transpose(rgb_w.astype(bf16), (0, 2, 1))
    top = jnp.pad(rgb_wT, ((0, 0), (0, 0), (0, Y_W - 3)))
    bot = jnp.pad(sigma_w.astype(bf16).reshape(M, 32, 1),
                  ((0, 0), (0, 0), (3, Y_W - 4)))
    yw = jnp.concatenate([top, bot], axis=1)
    yb = (jnp.pad(rgb_b, ((0, 0), (0, Y_W - 3)))
          + jnp.pad(sigma_b, ((0, 0), (3, Y_W - 4))))

    y = _c(xb, w0T, pts_b0, w1T, pts_b1, fwT, feat_b, vwT, view_b, yw, yb)
    res = _d(y, dest)

    rgb = res[:, 0:3].reshape(NRAYS, NSAMP, 3)
    sigma = res[:, 3:4].reshape(NRAYS, NSAMP, 1)
    return (rgb, sigma)


# speedup vs baseline: 1.4675x; 1.0032x over previous
import functools

import jax
import jax.numpy as jnp
from jax import lax
from jax.experimental import pallas as pl
from jax.experimental.pallas import tpu as pltpu
from jax.experimental.pallas import tpu_sc as plsc

GRID = 16
M = GRID * GRID * GRID
N = 32768
NRAYS = 1024
NSAMP = 32
C = 32
ROWS = M * C
L_XYZ = 10
L_DIR = 4
NW = 32
CH = N // NW
MB = 256
EP_W = 64
ED_W = 32
XB_W = EP_W + ED_W
Y_W = 16


def _sincos_rows(p, nlev):
    s = jnp.sin(p)
    c = jnp.cos(p)
    pieces = [p[0:3, :]]
    for i in range(nlev):
        pieces.append(s[0:3, :])
        pieces.append(c[0:3, :])
        if i + 1 < nlev:
            s, c = 2.0 * s * c, 1.0 - 2.0 * s * s
    return pieces


def _encode_body(x_ref, d_ref, o_ref):
    blk = x_ref.shape[1]
    rblk = d_ref.shape[1]
    pieces = _sincos_rows(x_ref[...], L_XYZ)
    off = 0
    for pc in pieces:
        o_ref[pl.ds(off, 3), :] = pc
        off += 3
    o_ref[pl.ds(off, EP_W - off), :] = jnp.zeros((EP_W - off, blk), jnp.float32)
    dpieces = _sincos_rows(d_ref[...], L_DIR)
    dpieces.append(jnp.zeros((ED_W - 3 * (1 + 2 * L_DIR), rblk), jnp.float32))
    dire = jnp.concatenate(dpieces, axis=0)
    diree = jnp.broadcast_to(dire[:, :, None], (ED_W, rblk, NSAMP))
    o_ref[pl.ds(EP_W, ED_W), :] = diree.reshape(ED_W, blk)


def _encode(x8, d8):
    blk = 4096
    rblk = blk // NSAMP
    return pl.pallas_call(
        _encode_body,
        grid=(N // blk,),
        in_specs=[pl.BlockSpec((8, blk), lambda i: (0, i)),
                  pl.BlockSpec((8, rblk), lambda i: (0, i))],
        out_specs=pl.BlockSpec((XB_W, blk), lambda i: (0, i)),
        out_shape=jax.ShapeDtypeStruct((XB_W, N), jnp.float32),
    )(x8, d8)


@functools.cache
def _mesh():
    return plsc.VectorSubcoreMesh(core_axis_name="c", subcore_axis_name="s")


_SC_PARAMS = pltpu.CompilerParams(use_tc_tiling_on_sc=False,
                                  needs_layout_passes=False)


def _wid():
    return lax.axis_index("c") * 16 + lax.axis_index("s")


IOTA16 = lambda: lax.iota(jnp.int32, 16)


def _rank_last(m16, scr16):
    iota = IOTA16()
    sk, sl = plsc.sort_key_val(m16, iota)
    scr16[...] = sk
    prevk = plsc.load_gather(scr16, [jnp.maximum(iota - 1, 0)])
    nextk = plsc.load_gather(scr16, [jnp.minimum(iota + 1, 15)])
    isb = (iota == 0) | (prevk != sk)
    start = plsc.cummax(jnp.where(isb, iota, 0))
    rank_sorted = iota - start
    last_sorted = (iota == 15) | (nextk != sk)
    plsc.store_scatter(scr16, [sl], rank_sorted)
    rank = scr16[...]
    plsc.store_scatter(scr16, [sl], last_sorted.astype(jnp.int32))
    last = scr16[...] != 0
    return rank, last


def _b1_body(xs_h, ys_h, zs_h, m_h, s1_h, tot_h, xv, yv, zv, mv, sv, hist,
             scr16):
    w = _wid()
    base = w * CH
    pltpu.sync_copy(xs_h.at[pl.ds(base, CH)], xv)
    pltpu.sync_copy(ys_h.at[pl.ds(base, CH)], yv)
    pltpu.sync_copy(zs_h.at[pl.ds(base, CH)], zv)

    def zero_body(t, _):
        hist[pl.ds(t * 16, 16)] = jnp.zeros((16,), jnp.int32)
        return 0
    lax.fori_loop(0, M // 16, zero_body, 0)

    def body(j, _):
        off = j * 16
        x16 = xv[pl.ds(off, 16)]
        y16 = yv[pl.ds(off, 16)]
        z16 = zv[pl.ds(off, 16)]
        def vox(v):
            return jnp.clip(v * float(GRID), 0.0, float(GRID - 1)).astype(jnp.int32)
        m16 = vox(x16) * (GRID * GRID) + vox(y16) * GRID + vox(z16)
        mv[pl.ds(off, 16)] = m16
        rank, lastm = _rank_last(m16, scr16)
        old = plsc.load_gather(hist, [m16])
        sv[pl.ds(off, 16)] = old + rank
        plsc.store_scatter(hist, [m16], old + rank + 1, mask=lastm)
        return 0
    lax.fori_loop(0, CH // 16, body, 0)

    pltpu.sync_copy(mv, m_h.at[w])
    pltpu.sync_copy(sv, s1_h.at[w])
    pltpu.sync_copy(hist, tot_h.at[w])


def _b1(xs, ys, zs):
    return pl.kernel(
        _b1_body,
        out_type=[
            jax.ShapeDtypeStruct((NW, CH), jnp.int32),
            jax.ShapeDtypeStruct((NW, CH), jnp.int32),
            jax.ShapeDtypeStruct((NW, M), jnp.int32),
        ],
        mesh=_mesh(),
        compiler_params=_SC_PARAMS,
        scratch_types=[
            pltpu.VMEM((CH,), jnp.float32),
            pltpu.VMEM((CH,), jnp.float32),
            pltpu.VMEM((CH,), jnp.float32),
            pltpu.VMEM((CH,), jnp.int32),
            pltpu.VMEM((CH,), jnp.int32),
            pltpu.VMEM((M,), jnp.int32),
            pltpu.VMEM((16,), jnp.int32),
        ],
    )(xs, ys, zs)


def _b15_body(tot_h, pre_h, buf, prebuf, sem):
    u = _wid()
    mpw = M // NW
    copies = [pltpu.make_async_copy(
        tot_h.at[w2, pl.ds(u * mpw, mpw)], buf.at[w2], sem)
        for w2 in range(NW)]
    for cp in copies:
        cp.start()
    for cp in copies:
        cp.wait()

    def g_body(g, _):
        def w_body(w2, run):
            prebuf[w2, pl.ds(g * 16, 16)] = run
            return run + plsc.load_gather(
                buf, [jnp.full((16,), w2, jnp.int32), IOTA16() + g * 16])
        lax.fori_loop(0, NW, w_body, jnp.zeros((16,), jnp.int32))
        return 0
    lax.fori_loop(0, mpw // 16, g_body, 0)

    pltpu.sync_copy(prebuf, pre_h.at[:, pl.ds(u * mpw, mpw)])


def _b15(tot):
    return pl.kernel(
        _b15_body,
        out_type=jax.ShapeDtypeStruct((NW, M), jnp.int32),
        mesh=_mesh(),
        compiler_params=_SC_PARAMS,
        scratch_types=[
            pltpu.VMEM((NW, M // NW), jnp.int32),
            pltpu.VMEM((NW, M // NW), jnp.int32),
            pltpu.SemaphoreType.DMA,
        ],
    )(tot)


def _b2_body(m_h, s1_h, pre_h, ep_h, xp_h, dest_h,
             acc, mv, sv, dv, epb, sem):
    w = _wid()
    pltpu.sync_copy(pre_h.at[w], acc)
    pltpu.sync_copy(m_h.at[w], mv)
    pltpu.sync_copy(s1_h.at[w], sv)

    def body(j, _):
        off = j * 16
        m16 = mv[pl.ds(off, 16)]
        slot = sv[pl.ds(off, 16)] + plsc.load_gather(acc, [m16])
        slot = jnp.minimum(slot, C - 1)
        dest16 = m16 * C + slot
        dv[j // 8, pl.ds((j % 8) * 16, 16)] = dest16
        return 0
    lax.fori_loop(0, CH // 16, body, 0)

    pltpu.sync_copy(dv, dest_h.at[w])
    base = w * CH
    pltpu.sync_copy(ep_h.at[pl.ds(base, CH), :], epb)
    copies = []
    for t in range(CH // 128):
        copies.append(pltpu.make_async_copy(
            epb.at[pl.ds(t * 128, 128), :], xp_h.at[dv.at[t]], sem))
    for cp in copies:
        cp.start()
    for cp in copies:
        cp.wait()


def _b2(m_all, s1, pre, ep):
    return pl.kernel(
        _b2_body,
        out_type=[
            jax.ShapeDtypeStruct((ROWS, XB_W), jnp.bfloat16),
            jax.ShapeDtypeStruct((NW, CH // 128, 128), jnp.int32),
        ],
        mesh=_mesh(),
        compiler_params=_SC_PARAMS,
        scratch_types=[
            pltpu.VMEM((M,), jnp.int32),
            pltpu.VMEM((CH,), jnp.int32),
            pltpu.VMEM((CH,), jnp.int32),
            pltpu.VMEM((CH // 128, 128), jnp.int32),
            pltpu.VMEM((CH, XB_W), jnp.bfloat16),
            pltpu.SemaphoreType.DMA,
        ],
    )(m_all, s1, pre, ep)


def _c_body(xb_ref, w0_ref, b0_ref, w1_ref, b1_ref, fw_ref, fb_ref,
            vw_ref, vb_ref, yw_ref, yb_ref, y_ref):
    f32 = jnp.float32
    bf16 = jnp.bfloat16
    dims_n = (((1,), (0,)), ((), ()))

    X = xb_ref[...]
    X64 = X[:, 0:EP_W]

    def rep(b_ref, width):
        b = b_ref[...]
        return jnp.broadcast_to(b[:, None, :], (MB, C, width)).reshape(MB * C, width)

    def layer_v(hval, w_ref2):
        wb = w_ref2[...]
        outs = [lax.dot_general(hval[i * C:(i + 1) * C, :], wb[i], dims_n,
                                preferred_element_type=f32)
                for i in range(MB)]
        return jnp.concatenate(outs, axis=0)

    H1 = jax.nn.relu(layer_v(X64, w0_ref) + rep(b0_ref, 32)).astype(bf16)
    H2 = jax.nn.relu(layer_v(H1, w1_ref) + rep(b1_ref, 32)).astype(bf16)
    FT = (layer_v(H2, fw_ref) + rep(fb_ref, 32)).astype(bf16)
    HVin = jnp.concatenate([FT, X[:, EP_W:XB_W]], axis=1)
    HV = jax.nn.relu(layer_v(HVin, vw_ref) + rep(vb_ref, 32)).astype(bf16)
    YC = jnp.concatenate([HV, H2], axis=1)
    Y = layer_v(YC, yw_ref) + rep(yb_ref, Y_W)
    y_ref[...] = Y


def _c(xb, w0, b0, w1, b1, fw, fb, vw, vb, yw, yb):
    nsteps = M // MB
    mspec = lambda shp: pl.BlockSpec((MB,) + shp, lambda i: (i,) + (0,) * len(shp))
    return pl.pallas_call(
        _c_body,
        grid=(nsteps,),
        in_specs=[
            pl.BlockSpec((MB * C, XB_W), lambda i: (i, 0)),
            mspec((64, 32)), mspec((32,)),
            mspec((32, 32)), mspec((32,)),
            mspec((32, 32)), mspec((32,)),
            mspec((64, 32)), mspec((32,)),
            mspec((64, Y_W)), mspec((Y_W,)),
        ],
        out_specs=pl.BlockSpec((MB * C, Y_W), lambda i: (i, 0)),
        out_shape=jax.ShapeDtypeStruct((ROWS, Y_W), jnp.float32),
    )(xb, w0, b0, w1, b1, fw, fb, vw, vb, yw, yb)


def _d_body(y_h, dest_h, out_h, dv, yb, sem):
    w = _wid()
    pltpu.sync_copy(dest_h.at[w], dv)
    for t in range(CH // 128):
        cp = pltpu.make_async_copy(y_h.at[dv.at[t]], yb, sem)
        cp.start()
        cp.wait()
        pltpu.sync_copy(yb, out_h.at[pl.ds(w * CH + t * 128, 128), :])


def _d(y, dest):
    return pl.kernel(
        _d_body,
        out_type=jax.ShapeDtypeStruct((N, Y_W), jnp.float32),
        mesh=_mesh(),
        compiler_params=_SC_PARAMS,
        scratch_types=[
            pltpu.VMEM((CH // 128, 128), jnp.int32),
            pltpu.VMEM((128, Y_W), jnp.float32),
            pltpu.SemaphoreType.DMA,
        ],
    )(y, dest)


def kernel(pts, viewdirs, pts_w0, pts_b0, pts_w1, pts_b1, feat_w, feat_b,
           sigma_w, sigma_b, view_w, view_b, rgb_w, rgb_b):
    bf16 = jnp.bfloat16
    pts_flat = pts.reshape(N, 3)
    x8 = jnp.pad(pts_flat.T, ((0, 5), (0, 0)))
    d8 = jnp.pad(viewdirs.T, ((0, 5), (0, 0)))

    eT = _encode(x8, d8)
    ep = eT.T.astype(bf16)

    m_all, s1, tot = _b1(x8[0], x8[1], x8[2])
    pre = _b15(tot)
    xb, dest = _b2(m_all, s1, pre, ep)

    def tpad(wm, rows):
        wt = jnp.transpose(wm.astype(bf16), (0, 2, 1))
        return jnp.pad(wt, ((0, 0), (0, rows - wt.shape[1]), (0, 0)))
    w0T = tpad(pts_w0, 64)
    w1T = tpad(pts_w1, 32)
    fwT = tpad(feat_w, 32)
    vwT = tpad(view_w, 64)

    rgb_wT = jnp.transpose(rgb_w.astype(bf16), (0, 2, 1))
    top = jnp.pad(rgb_wT, ((0, 0), (0, 0), (0, Y_W - 3)))
    bot = jnp.pad(sigma_w.astype(bf16).reshape(M, 32, 1),
                  ((0, 0), (0, 0), (3, Y_W - 4)))
    yw = jnp.concatenate([top, bot], axis=1)
    yb = (jnp.pad(rgb_b, ((0, 0), (0, Y_W - 3)))
          + jnp.pad(sigma_b, ((0, 0), (3, Y_W - 4))))

    y = _c(xb, w0T, pts_b0, w1T, pts_b1, fwT, feat_b, vwT, view_b, yw, yb)
    res = _d(y, dest)

    rgb = res[:, 0:3].reshape(NRAYS, NSAMP, 3)
    sigma = res[:, 3:4].reshape(NRAYS, NSAMP, 1)
    return (rgb, sigma)
